# Initial kernel scaffold; baseline (speedup 1.0000x reference)
import jax
import jax.numpy as jnp
from jax.experimental import pallas as pl


def kernel(extended_sub_adj, sub_feat, M, pairs, W1, b1, W2, b2, top_k):
    raise NotImplementedError("write your pallas kernel here")


# speedup vs baseline: 3.4524x; 3.4524x over previous
import jax
import jax.numpy as jnp
from jax import lax
from jax.experimental import pallas as pl
from jax.experimental.pallas import tpu as pltpu
from jax.experimental.pallas import tpu_sc as plsc

N = 4096
NHID = 64
NCLASS = 8
NMASK = 100000
TOPK = 1024
NP = 100352
NROWS = NP // 128
ONE_BITS = 0x3F800000
MAXOV = 1024
HALF_CODES = 1 << 23
HASH_C = -1640531527

NSUB = 16
PAIRS_PER_TILE = NP // NSUB
CHUNK = 128
CHUNKS_PER_TILE = PAIRS_PER_TILE // CHUNK
SC_MESH = dict(core_axis_name="c", subcore_axis_name="s")


def _k1_body(m_ref, src_ref, dst_ref, feat_ref, w1_ref, tk_ref,
             act_ref, code_ref, rcode_ref, x1_ref):
    m = m_ref[...]
    tm = jnp.tanh(m)
    a = jnp.abs(tm)
    ri = lax.broadcasted_iota(jnp.int32, (NROWS, 128), 0)
    ci = lax.broadcasted_iota(jnp.int32, (NROWS, 128), 1)
    gi = ri * 128 + ci
    valid = gi < NMASK
    key = jnp.where(valid, lax.bitcast_convert_type(a, jnp.int32),
                    jnp.int32(-1))
    k = jnp.clip(tk_ref[0, 0], 0, TOPK)

    def vstep(_, lohi):
        lo, hi = lohi
        mid = lax.div(lo + hi, 2)
        c = jnp.sum((key > mid).astype(jnp.int32))
        take_hi = c < k
        return (jnp.where(take_hi, lo, mid + 1), jnp.where(take_hi, mid, hi))

    thr, _ = lax.fori_loop(0, 31, vstep, (jnp.int32(0), jnp.int32(ONE_BITS)))
    eq = key == thr
    n_gt = jnp.sum((key > thr).astype(jnp.int32))
    tn = k - n_gt

    def istep(_, lohi):
        lo2, hi2 = lohi
        mid = lax.div(lo2 + hi2, 2)
        h = jnp.sum((eq & (gi < mid)).astype(jnp.int32))
        ge = h >= tn
        return (jnp.where(ge, lo2, mid + 1), jnp.where(ge, mid, hi2))

    cut, _ = lax.fori_loop(0, 17, istep, (jnp.int32(0), jnp.int32(1 << 17)))
    keep = (key > thr) | (eq & (gi < cut))
    act_ref[...] = jnp.where(
        keep & (tm > 0.5), jnp.int32(1),
        jnp.where(keep & (tm < -0.5), jnp.int32(-1), jnp.int32(0)))
    s = src_ref[...]
    d = dst_ref[...]
    code_ref[...] = lax.shift_left(s, 12) + d
    rcode_ref[...] = lax.shift_left(d, 12) + s
    x1_ref[...] = jnp.dot(feat_ref[...], w1_ref[...],
                          preferred_element_type=jnp.float32)


def _k1(m_p, src_p, dst_p, sub_feat, w1, tk):
    return pl.pallas_call(
        _k1_body,
        in_specs=[pl.BlockSpec(memory_space=pltpu.VMEM)] * 5
        + [pl.BlockSpec(memory_space=pltpu.SMEM)],
        out_specs=[pl.BlockSpec(memory_space=pltpu.VMEM)] * 4,
        out_shape=[
            jax.ShapeDtypeStruct((NROWS, 128), jnp.int32),
            jax.ShapeDtypeStruct((NROWS, 128), jnp.int32),
            jax.ShapeDtypeStruct((NROWS, 128), jnp.int32),
            jax.ShapeDtypeStruct((N, NHID), jnp.float32),
        ],
    )(m_p, src_p, dst_p, sub_feat, w1, tk)


def _k2_body(codes_hbm, rcodes_hbm,
             cntc_hbm, cntr_hbm, dsum_hbm,
             cvec, rvec, idxb, valb, gbuf, cnts_loc, zbuf,
             cnt_tab, dup_tab):
    c = lax.axis_index("c")
    s = lax.axis_index("s")
    lanes = jnp.arange(16, dtype=jnp.int32)
    tile_base = s * PAIRS_PER_TILE

    def own_word(v, gidx):
        m = (gidx < NMASK) & (lax.shift_right_logical(v, 23) == c)
        w = lax.shift_right_logical(v & (HALF_CODES - 1), 3)
        return m, w

    for i in range(256):
        zbuf[pl.ds(i * 16, 16)] = jnp.zeros((16,), jnp.int32)

    def zstep(j, _):
        pltpu.sync_copy(zbuf, cnt_tab.at[pl.ds(s * 65536 + j * 4096, 4096)])
        return 0

    lax.fori_loop(0, 16, zstep, 0)
    pltpu.sync_copy(zbuf, dup_tab.at[pl.ds(s * 4096, 4096)])
    plsc.subcore_barrier()

    def sstep(j, _):
        base = tile_base + j * CHUNK
        pltpu.sync_copy(codes_hbm.at[pl.ds(base, CHUNK)], cvec)
        for kk in range(8):
            v = cvec[pl.ds(kk * 16, 16)]
            m, w = own_word(v, base + kk * 16 + lanes)
            addv = lax.shift_left(jnp.int32(1), (v & 7) * 4)
            idxb[pl.ds(kk * 16, 16)] = jnp.where(m, w, 0)
            valb[pl.ds(kk * 16, 16)] = jnp.where(m, addv, 0)
        pltpu.sync_copy(valb, cnt_tab.at[idxb], add=True)
        return 0

    lax.fori_loop(0, CHUNKS_PER_TILE, sstep, 0)
    plsc.subcore_barrier()

    def gstep(j, _):
        base = tile_base + j * CHUNK
        pltpu.sync_copy(codes_hbm.at[pl.ds(base, CHUNK)], cvec)
        pltpu.sync_copy(rcodes_hbm.at[pl.ds(base, CHUNK)], rvec)
        for kk in range(8):
            v = cvec[pl.ds(kk * 16, 16)]
            m, w = own_word(v, base + kk * 16 + lanes)
            idxb[pl.ds(kk * 16, 16)] = jnp.where(m, w, 0)
        pltpu.sync_copy(cnt_tab.at[idxb], gbuf)
        for kk in range(8):
            v = cvec[pl.ds(kk * 16, 16)]
            m, _w = own_word(v, base + kk * 16 + lanes)
            g = gbuf[pl.ds(kk * 16, 16)]
            cnt = lax.shift_right_logical(g, (v & 7) * 4) & 15
            cnt = jnp.where(m, cnt, 0)
            valb[pl.ds(kk * 16, 16)] = cnt
            cnts_loc[pl.ds(j * CHUNK + kk * 16, 16)] = cnt
        pltpu.sync_copy(valb, cntc_hbm.at[c].at[pl.ds(base, CHUNK)])
        for kk in range(8):
            v = rvec[pl.ds(kk * 16, 16)]
            m, w = own_word(v, base + kk * 16 + lanes)
            idxb[pl.ds(kk * 16, 16)] = jnp.where(m, w, 0)
        pltpu.sync_copy(cnt_tab.at[idxb], gbuf)
        for kk in range(8):
            v = rvec[pl.ds(kk * 16, 16)]
            m, _w = own_word(v, base + kk * 16 + lanes)
            g = gbuf[pl.ds(kk * 16, 16)]
            cnt = lax.shift_right_logical(g, (v & 7) * 4) & 15
            valb[pl.ds(kk * 16, 16)] = jnp.where(m, cnt, 0)
        pltpu.sync_copy(valb, cntr_hbm.at[c].at[pl.ds(base, CHUNK)])
        return 0

    lax.fori_loop(0, CHUNKS_PER_TILE, gstep, 0)

    def dstep(j, _):
        base = tile_base + j * CHUNK
        pltpu.sync_copy(codes_hbm.at[pl.ds(base, CHUNK)], cvec)
        for kk in range(8):
            v = cvec[pl.ds(kk * 16, 16)]
            gidx = base + kk * 16 + lanes
            m, _w = own_word(v, gidx)
            m = m & (cnts_loc[pl.ds(j * CHUNK + kk * 16, 16)] >= 2)
            slot = lax.shift_right_logical(v * HASH_C, 16) & 0xFFFF
            idxb[pl.ds(kk * 16, 16)] = jnp.where(m, slot, 0)
            valb[pl.ds(kk * 16, 16)] = jnp.where(m, gidx, 0)
        pltpu.sync_copy(valb, dup_tab.at[idxb], add=True)
        return 0

    lax.fori_loop(0, CHUNKS_PER_TILE, dstep, 0)
    plsc.subcore_barrier()

    def estep(j, _):
        base = tile_base + j * CHUNK
        pltpu.sync_copy(codes_hbm.at[pl.ds(base, CHUNK)], cvec)
        for kk in range(8):
            v = cvec[pl.ds(kk * 16, 16)]
            m, _w = own_word(v, base + kk * 16 + lanes)
            m = m & (cnts_loc[pl.ds(j * CHUNK + kk * 16, 16)] >= 2)
            slot = lax.shift_right_logical(v * HASH_C, 16) & 0xFFFF
            idxb[pl.ds(kk * 16, 16)] = jnp.where(m, slot, 0)
        pltpu.sync_copy(dup_tab.at[idxb], gbuf)
        for kk in range(8):
            v = cvec[pl.ds(kk * 16, 16)]
            m, _w = own_word(v, base + kk * 16 + lanes)
            m = m & (cnts_loc[pl.ds(j * CHUNK + kk * 16, 16)] >= 2)
            valb[pl.ds(kk * 16, 16)] = jnp.where(m, gbuf[pl.ds(kk * 16, 16)], 0)
        pltpu.sync_copy(valb, dsum_hbm.at[c].at[pl.ds(base, CHUNK)])
        return 0

    lax.fori_loop(0, CHUNKS_PER_TILE, estep, 0)


def _k2(codes, rcodes):
    fn = pl.kernel(
        _k2_body,
        mesh=plsc.VectorSubcoreMesh(**SC_MESH),
        out_type=[
            jax.ShapeDtypeStruct((2, NP), jnp.int32),
            jax.ShapeDtypeStruct((2, NP), jnp.int32),
            jax.ShapeDtypeStruct((2, NP), jnp.int32),
        ],
        scratch_types=[
            pltpu.VMEM((CHUNK,), jnp.int32),
            pltpu.VMEM((CHUNK,), jnp.int32),
            pltpu.VMEM((CHUNK,), jnp.int32),
            pltpu.VMEM((CHUNK,), jnp.int32),
            pltpu.VMEM((CHUNK,), jnp.int32),
            pltpu.VMEM((PAIRS_PER_TILE,), jnp.int32),
            pltpu.VMEM((4096,), jnp.int32),
            pltpu.VMEM_SHARED((1 << 20,), jnp.int32),
            pltpu.VMEM_SHARED((1 << 16,), jnp.int32),
        ],
    )
    return fn(codes, rcodes)


def _k3b_body(adj_ref, out_ref):
    out_ref[...] = jnp.sum(adj_ref[...], axis=1).reshape(1, 1, 128)


def _k3b(adj):
    return pl.pallas_call(
        _k3b_body,
        grid=(32,),
        in_specs=[pl.BlockSpec((128, N), lambda i: (i, 0))],
        out_specs=pl.BlockSpec((1, 1, 128), lambda i: (i, 0, 0)),
        out_shape=jax.ShapeDtypeStruct((32, 1, 128), jnp.float32),
    )(adj)


def _k3a_body(act_ref, code_ref, rcode_ref, cc0_ref, cc1_ref, cr0_ref,
              cr1_ref, ds0_ref, ds1_ref, packed_ref, dest_ref, nn_ref):
    act = act_ref[...]
    cnt = cc0_ref[...] + cc1_ref[...]
    cntr = cr0_ref[...] + cr1_ref[...]
    dsum = ds0_ref[...] + ds1_ref[...]
    ri = lax.broadcasted_iota(jnp.int32, (NROWS, 128), 0)
    ci = lax.broadcasted_iota(jnp.int32, (NROWS, 128), 1)
    gi = ri * 128 + ci
    surv_a = (cnt == 1) | ((cnt == 2) & (2 * gi > dsum))
    cand = act != 0
    tag = lax.shift_left((act > 0).astype(jnp.int32), 24) + jnp.int32(1 << 25)
    surv0 = cand & surv_a & (cntr == 0)
    surv1 = cand & surv_a
    packed_ref[0:NROWS, :] = jnp.where(surv0, code_ref[...] + tag, 0)
    packed_ref[NROWS:2 * NROWS, :] = jnp.where(surv1, rcode_ref[...] + tag, 0)

    tri = (lax.broadcasted_iota(jnp.int32, (NROWS, NROWS), 0)
           > lax.broadcasted_iota(jnp.int32, (NROWS, NROWS), 1)
           ).astype(jnp.float32)
    lt = (lax.broadcasted_iota(jnp.int32, (128, 128), 0)
          <= lax.broadcasted_iota(jnp.int32, (128, 128), 1)
          ).astype(jnp.float32)

    def prefix(sv):
        v = sv.astype(jnp.float32)
        incl = jnp.dot(v, lt, preferred_element_type=jnp.float32)
        rowtot = incl[:, 127:128]
        row_off = jnp.dot(tri, rowtot, preferred_element_type=jnp.float32)
        dest = (row_off + incl - v).astype(jnp.int32)
        n = jnp.sum(sv.astype(jnp.int32))
        return dest, n

    d0, n0 = prefix(surv0)
    d1, n1 = prefix(surv1)
    dest_ref[0:NROWS, :] = d0
    dest_ref[NROWS:2 * NROWS, :] = d1
    nn_ref[0, 0] = n0
    nn_ref[1, 0] = n1


def _k3a(act, code, rcode, cc0, cc1, cr0, cr1, ds0, ds1):
    return pl.pallas_call(
        _k3a_body,
        in_specs=[pl.BlockSpec(memory_space=pltpu.VMEM)] * 9,
        out_specs=[
            pl.BlockSpec(memory_space=pltpu.VMEM),
            pl.BlockSpec(memory_space=pltpu.VMEM),
            pl.BlockSpec(memory_space=pltpu.SMEM),
        ],
        out_shape=[
            jax.ShapeDtypeStruct((2 * NROWS, 128), jnp.int32),
            jax.ShapeDtypeStruct((2 * NROWS, 128), jnp.int32),
            jax.ShapeDtypeStruct((16, 1), jnp.int32),
        ],
    )(act, code, rcode, cc0, cc1, cr0, cr1, ds0, ds1)


def _k4_body(packed_hbm, dest_hbm, nn_hbm, adj_hbm,
             ovrow_hbm, ovcol_hbm, ovdelta_hbm, degd_hbm,
             pvec, dvec, idxb, valb, idx64, rowb, colb, deltab, oldb,
             nbuf, zbi, zbf, ov_sp, degd_sp):
    c = lax.axis_index("c")
    s = lax.axis_index("s")
    lanes = jnp.arange(16, dtype=jnp.int32)
    tile_base = s * PAIRS_PER_TILE

    for i in range(16):
        zbi[pl.ds(i * 16, 16)] = jnp.zeros((16,), jnp.int32)
        zbf[pl.ds(i * 16, 16)] = jnp.zeros((16,), jnp.float32)
    pltpu.sync_copy(zbi.at[pl.ds(0, 64)], ov_sp.at[pl.ds(s * 64, 64)])
    pltpu.sync_copy(zbf, degd_sp.at[pl.ds(s * 256, 256)])
    pltpu.sync_copy(nn_hbm, nbuf)
    plsc.subcore_barrier()

    def pstep(j, _):
        base = tile_base + j * CHUNK
        pltpu.sync_copy(packed_hbm.at[c].at[pl.ds(base, CHUNK)], pvec)
        pltpu.sync_copy(dest_hbm.at[c].at[pl.ds(base, CHUNK)], dvec)
        for kk in range(8):
            v = pvec[pl.ds(kk * 16, 16)]
            dd = dvec[pl.ds(kk * 16, 16)]
            m = (lax.shift_right_logical(v, 25) & 1) == 1
            idxb[pl.ds(kk * 16, 16)] = jnp.where(m, dd, 0)
            valb[pl.ds(kk * 16, 16)] = jnp.where(m, v, 0)
        pltpu.sync_copy(valb, ov_sp.at[idxb], add=True)
        return 0

    lax.fori_loop(0, CHUNKS_PER_TILE, pstep, 0)
    plsc.subcore_barrier()

    slot_base = s * 64
    pltpu.sync_copy(ov_sp.at[pl.ds(slot_base, 64)], pvec.at[pl.ds(0, 64)])
    nv = nbuf[pl.ds(0, 16)]
    n_c = jnp.where(c == 0, nv[0], nv[1])
    for kk in range(4):
        v = pvec[pl.ds(kk * 16, 16)]
        live = (slot_base + kk * 16 + lanes) < n_c
        idx64[pl.ds(kk * 16, 16)] = jnp.where(live, v & 0xFFFFFF, 0)
    pltpu.sync_copy(adj_hbm.at[idx64], oldb)
    for kk in range(4):
        v = pvec[pl.ds(kk * 16, 16)]
        live = (slot_base + kk * 16 + lanes) < n_c
        codev = v & 0xFFFFFF
        newv = (lax.shift_right_logical(v, 24) & 1).astype(jnp.float32)
        delta = jnp.where(live, newv - oldb[pl.ds(kk * 16, 16)], 0.0)
        rowb[pl.ds(kk * 16, 16)] = jnp.where(
            live, lax.shift_right_logical(codev, 12), 0)
        colb[pl.ds(kk * 16, 16)] = jnp.where(live, codev & 4095, 0)
        deltab[pl.ds(kk * 16, 16)] = delta
    pltpu.sync_copy(rowb, ovrow_hbm.at[c].at[pl.ds(slot_base, 64)])
    pltpu.sync_copy(colb, ovcol_hbm.at[c].at[pl.ds(slot_base, 64)])
    pltpu.sync_copy(deltab, ovdelta_hbm.at[c].at[pl.ds(slot_base, 64)])
    pltpu.sync_copy(deltab, degd_sp.at[rowb], add=True)
    plsc.subcore_barrier()

    pltpu.sync_copy(degd_sp.at[pl.ds(s * 256, 256)],
                    degd_hbm.at[c].at[pl.ds(s * 256, 256)])


def _k4(packed, dest, nn, adj_flat):
    fn = pl.kernel(
        _k4_body,
        mesh=plsc.VectorSubcoreMesh(**SC_MESH),
        out_type=[
            jax.ShapeDtypeStruct((2, MAXOV), jnp.int32),
            jax.ShapeDtypeStruct((2, MAXOV), jnp.int32),
            jax.ShapeDtypeStruct((2, MAXOV), jnp.float32),
            jax.ShapeDtypeStruct((2, N), jnp.float32),
        ],
        scratch_types=[
            pltpu.VMEM((CHUNK,), jnp.int32),
            pltpu.VMEM((CHUNK,), jnp.int32),
            pltpu.VMEM((CHUNK,), jnp.int32),
            pltpu.VMEM((CHUNK,), jnp.int32),
            pltpu.VMEM((64,), jnp.int32),
            pltpu.VMEM((64,), jnp.int32),
            pltpu.VMEM((64,), jnp.int32),
            pltpu.VMEM((64,), jnp.float32),
            pltpu.VMEM((64,), jnp.float32),
            pltpu.VMEM((16,), jnp.int32),
            pltpu.VMEM((256,), jnp.int32),
            pltpu.VMEM((256,), jnp.float32),
            pltpu.VMEM_SHARED((MAXOV,), jnp.int32),
            pltpu.VMEM_SHARED((N,), jnp.float32),
        ],
    )
    return fn(packed, dest, nn, adj_flat)


def _k5_body(bd_ref, dd0_ref, dd1_ref, x1_ref, d_ref, xs1_ref):
    deg = bd_ref[...] + dd0_ref[...] + dd1_ref[...] + 1.0
    d = lax.rsqrt(jnp.maximum(deg, 1e-12))
    d_ref[...] = d
    xs1_ref[:, 0:NHID] = d * x1_ref[...]
    xs1_ref[:, NHID:128] = jnp.zeros((N, 128 - NHID), jnp.float32)


def _k5(base_deg, dd0, dd1, x1):
    return pl.pallas_call(
        _k5_body,
        in_specs=[pl.BlockSpec(memory_space=pltpu.VMEM)] * 4,
        out_specs=[pl.BlockSpec(memory_space=pltpu.VMEM)] * 2,
        out_shape=[
            jax.ShapeDtypeStruct((N, 1), jnp.float32),
            jax.ShapeDtypeStruct((N, 128), jnp.float32),
        ],
    )(base_deg, dd0, dd1, x1)


def _urow_body(width, ovrow_hbm, ovcol_hbm, ovdelta_hbm, tab_hbm, out_hbm,
               rowb, colb, deltab, rows_v, zrow, u_sp):
    c = lax.axis_index("c")
    s = lax.axis_index("s")
    rows_per_tile = N // NSUB

    for i in range(16):
        for kk in range(width // 16):
            zrow[i, pl.ds(kk * 16, 16)] = jnp.zeros((16,), jnp.float32)

    def zstep(j, _):
        pltpu.sync_copy(zrow, u_sp.at[pl.ds(s * rows_per_tile + j * 16, 16)])
        return 0

    lax.fori_loop(0, rows_per_tile // 16, zstep, 0)
    base = s * 64
    pltpu.sync_copy(ovrow_hbm.at[c].at[pl.ds(base, 64)], rowb)
    pltpu.sync_copy(ovcol_hbm.at[c].at[pl.ds(base, 64)], colb)
    pltpu.sync_copy(ovdelta_hbm.at[c].at[pl.ds(base, 64)], deltab)
    plsc.subcore_barrier()

    pltpu.sync_copy(tab_hbm.at[colb], rows_v)
    dvs = [deltab[pl.ds(q * 16, 16)] for q in range(4)]
    for i in range(64):
        dsc = dvs[i // 16][i % 16]
        for kk in range(width // 16):
            rows_v[i, pl.ds(kk * 16, 16)] = rows_v[i, pl.ds(kk * 16, 16)] * dsc
    pltpu.sync_copy(rows_v, u_sp.at[rowb], add=True)
    plsc.subcore_barrier()

    pltpu.sync_copy(u_sp.at[pl.ds(s * rows_per_tile, rows_per_tile)],
                    out_hbm.at[c].at[pl.ds(s * rows_per_tile, rows_per_tile)])


def _k_urow(width, ovrow, ovcol, ovdelta, tab):
    def body(*refs):
        _urow_body(width, *refs)

    fn = pl.kernel(
        body,
        mesh=plsc.VectorSubcoreMesh(**SC_MESH),
        out_type=[jax.ShapeDtypeStruct((2, N, width), jnp.float32)],
        scratch_types=[
            pltpu.VMEM((64,), jnp.int32),
            pltpu.VMEM((64,), jnp.int32),
            pltpu.VMEM((64,), jnp.float32),
            pltpu.VMEM((64, width), jnp.float32),
            pltpu.VMEM((16, width), jnp.float32),
            pltpu.VMEM_SHARED((N, width), jnp.float32),
        ],
    )
    return fn(ovrow, ovcol, ovdelta, tab)[0]


def _k7_body(adj_ref, xs1_ref, u1a_ref, u1b_ref, d_ref, b1_ref, w2_ref,
             gs_ref):
    i = pl.program_id(0)
    y = jnp.dot(adj_ref[...], xs1_ref[...], preferred_element_type=jnp.float32)
    rows = xs1_ref[pl.ds(i * 256, 256), :]
    t = d_ref[...] * (y + u1a_ref[...] + u1b_ref[...] + rows) + b1_ref[...]
    h = jnp.maximum(t, 0.0)
    gs_ref[...] = d_ref[...] * jnp.dot(h, w2_ref[...],
                                       preferred_element_type=jnp.float32)


def _k7(adj, xs1, u1a, u1b, d, b1, w2p):
    return pl.pallas_call(
        _k7_body,
        grid=(16,),
        in_specs=[
            pl.BlockSpec((256, N), lambda i: (i, 0)),
            pl.BlockSpec((N, 128), lambda i: (0, 0)),
            pl.BlockSpec((256, 128), lambda i: (i, 0)),
            pl.BlockSpec((256, 128), lambda i: (i, 0)),
            pl.BlockSpec((256, 1), lambda i: (i, 0)),
            pl.BlockSpec((1, 128), lambda i: (0, 0)),
            pl.BlockSpec((128, 128), lambda i: (0, 0)),
        ],
        out_specs=pl.BlockSpec((256, 128), lambda i: (i, 0)),
        out_shape=jax.ShapeDtypeStruct((N, 128), jnp.float32),
    )(adj, xs1, u1a, u1b, d, b1, w2p)


def _k9_body(adj_ref, gs_ref, u2a_ref, u2b_ref, d_ref, b2_ref, out_ref):
    i = pl.program_id(0)
    y = jnp.dot(adj_ref[...], gs_ref[...], preferred_element_type=jnp.float32)
    rows = gs_ref[pl.ds(i * 256, 256), :]
    out_ref[...] = (d_ref[...] * (y + u2a_ref[...] + u2b_ref[...] + rows)
                    + b2_ref[...])


def _k9(adj, gs, u2a, u2b, d, b2p):
    return pl.pallas_call(
        _k9_body,
        grid=(16,),
        in_specs=[
            pl.BlockSpec((256, N), lambda i: (i, 0)),
            pl.BlockSpec((N, 128), lambda i: (0, 0)),
            pl.BlockSpec((256, 128), lambda i: (i, 0)),
            pl.BlockSpec((256, 128), lambda i: (i, 0)),
            pl.BlockSpec((256, 1), lambda i: (i, 0)),
            pl.BlockSpec((1, 128), lambda i: (0, 0)),
        ],
        out_specs=pl.BlockSpec((256, 128), lambda i: (i, 0)),
        out_shape=jax.ShapeDtypeStruct((N, 128), jnp.float32),
    )(adj, gs, u2a, u2b, d, b2p)


def kernel(extended_sub_adj, sub_feat, M, pairs, W1, b1, W2, b2, top_k):
    pad = NP - NMASK
    m_p = jnp.pad(M, (0, pad)).reshape(NROWS, 128)
    src = jnp.pad(pairs[:, 0], (0, pad)).reshape(NROWS, 128)
    dst = jnp.pad(pairs[:, 1], (0, pad)).reshape(NROWS, 128)
    tk = jnp.asarray(top_k, jnp.int32).reshape(1, 1)

    act, code, rcode, x1 = _k1(m_p, src, dst, sub_feat, W1, tk)
    cntc, cntr, dsum = _k2(code.reshape(NP), rcode.reshape(NP))
    base_deg = _k3b(extended_sub_adj).reshape(N, 1)
    packed, dest, nn = _k3a(
        act, code, rcode,
        cntc[0].reshape(NROWS, 128), cntc[1].reshape(NROWS, 128),
        cntr[0].reshape(NROWS, 128), cntr[1].reshape(NROWS, 128),
        dsum[0].reshape(NROWS, 128), dsum[1].reshape(NROWS, 128))
    ovrow, ovcol, ovdelta, degd = _k4(
        packed.reshape(2, NP), dest.reshape(2, NP), nn.reshape(16),
        extended_sub_adj.reshape(N * N))
    d, xs1 = _k5(base_deg, degd[0].reshape(N, 1), degd[1].reshape(N, 1), x1)
    u1 = _k_urow(128, ovrow, ovcol, ovdelta, xs1)
    w2p = jnp.pad(W2, ((0, 128 - NHID), (0, 128 - NCLASS)))
    b1p = jnp.pad(b1, (0, 128 - NHID)).reshape(1, 128)
    b2p = jnp.pad(b2, (0, 128 - NCLASS)).reshape(1, 128)
    gs = _k7(extended_sub_adj, xs1, u1[0], u1[1], d, b1p, w2p)
    u2 = _k_urow(128, ovrow, ovcol, ovdelta, gs)
    out = _k9(extended_sub_adj, gs, u2[0], u2[1], d, b2p)
    return out[:, :NCLASS]


# speedup vs baseline: 3.4895x; 1.0107x over previous
import jax
import jax.numpy as jnp
from jax import lax
from jax.experimental import pallas as pl
from jax.experimental.pallas import tpu as pltpu
from jax.experimental.pallas import tpu_sc as plsc

N = 4096
NHID = 64
NCLASS = 8
NMASK = 100000
TOPK = 1024
NP = 100352
NROWS = NP // 128
ONE_BITS = 0x3F800000
MAXOV = 1024
HALF_CODES = 1 << 23
HASH_C = -1640531527

NSUB = 16
PAIRS_PER_TILE = NP // NSUB
CHUNK = 128
CHUNKS_PER_TILE = PAIRS_PER_TILE // CHUNK
SC_MESH = dict(core_axis_name="c", subcore_axis_name="s")


def _k1_body(m_ref, src_ref, dst_ref, feat_ref, w1_ref, tk_ref,
             act_ref, code_ref, rcode_ref, x1_ref):
    m = m_ref[...]
    tm = jnp.tanh(m)
    a = jnp.abs(tm)
    ri = lax.broadcasted_iota(jnp.int32, (NROWS, 128), 0)
    ci = lax.broadcasted_iota(jnp.int32, (NROWS, 128), 1)
    gi = ri * 128 + ci
    valid = gi < NMASK
    key = jnp.where(valid, lax.bitcast_convert_type(a, jnp.int32),
                    jnp.int32(-1))
    k = jnp.clip(tk_ref[0, 0], 0, TOPK)

    def vstep(_, lohi):
        lo, hi = lohi
        mid = lax.div(lo + hi, 2)
        c = jnp.sum((key > mid).astype(jnp.int32))
        take_hi = c < k
        return (jnp.where(take_hi, lo, mid + 1), jnp.where(take_hi, mid, hi))

    thr, _ = lax.fori_loop(0, 31, vstep, (jnp.int32(0), jnp.int32(ONE_BITS)))
    eq = key == thr
    n_gt = jnp.sum((key > thr).astype(jnp.int32))
    tn = k - n_gt

    def istep(_, lohi):
        lo2, hi2 = lohi
        mid = lax.div(lo2 + hi2, 2)
        h = jnp.sum((eq & (gi < mid)).astype(jnp.int32))
        ge = h >= tn
        return (jnp.where(ge, lo2, mid + 1), jnp.where(ge, mid, hi2))

    cut, _ = lax.fori_loop(0, 17, istep, (jnp.int32(0), jnp.int32(1 << 17)))
    keep = (key > thr) | (eq & (gi < cut))
    act_ref[...] = jnp.where(
        keep & (tm > 0.5), jnp.int32(1),
        jnp.where(keep & (tm < -0.5), jnp.int32(-1), jnp.int32(0)))
    s = src_ref[...]
    d = dst_ref[...]
    code_ref[...] = lax.shift_left(s, 12) + d
    rcode_ref[...] = lax.shift_left(d, 12) + s
    x1_ref[...] = jnp.dot(feat_ref[...], w1_ref[...],
                          preferred_element_type=jnp.float32)


def _k1(m_p, src_p, dst_p, sub_feat, w1, tk):
    return pl.pallas_call(
        _k1_body,
        in_specs=[pl.BlockSpec(memory_space=pltpu.VMEM)] * 5
        + [pl.BlockSpec(memory_space=pltpu.SMEM)],
        out_specs=[pl.BlockSpec(memory_space=pltpu.VMEM)] * 4,
        out_shape=[
            jax.ShapeDtypeStruct((NROWS, 128), jnp.int32),
            jax.ShapeDtypeStruct((NROWS, 128), jnp.int32),
            jax.ShapeDtypeStruct((NROWS, 128), jnp.int32),
            jax.ShapeDtypeStruct((N, NHID), jnp.float32),
        ],
    )(m_p, src_p, dst_p, sub_feat, w1, tk)


def _k2_body(codes_hbm, rcodes_hbm,
             cntc_hbm, cntr_hbm, dsum_hbm,
             codes_all, rcodes_all, idx_all, val_all, gbuf_all,
             cnts_all, outp, zbuf, sem,
             cnt_tab, dup_tab):
    c = lax.axis_index("c")
    s = lax.axis_index("s")
    lanes = jnp.arange(16, dtype=jnp.int32)
    tile_base = s * PAIRS_PER_TILE
    nch = CHUNKS_PER_TILE

    def own_word(v, gidx):
        m = (gidx < NMASK) & (lax.shift_right_logical(v, 23) == c)
        w = lax.shift_right_logical(v & (HALF_CODES - 1), 3)
        return m, w


    h_c = pltpu.async_copy(
        codes_hbm.at[pl.ds(tile_base, PAIRS_PER_TILE)], codes_all, sem)
    h_r = pltpu.async_copy(
        rcodes_hbm.at[pl.ds(tile_base, PAIRS_PER_TILE)], rcodes_all, sem)

    def zfill(j, _):
        zbuf[pl.ds(j * 16, 16)] = jnp.zeros((16,), jnp.int32)
        return 0

    lax.fori_loop(0, 1024, zfill, 0)
    for j in range(4):
        pltpu.sync_copy(zbuf, cnt_tab.at[pl.ds(s * 65536 + j * 16384, 16384)])
    pltpu.sync_copy(zbuf.at[pl.ds(0, 4096)], dup_tab.at[pl.ds(s * 4096, 4096)])
    h_c.wait()
    h_r.wait()
    plsc.subcore_barrier()

    def scomp(j, _):
        for kk in range(8):
            o = j * CHUNK + kk * 16
            v = codes_all[pl.ds(o, 16)]
            m, w = own_word(v, tile_base + o + lanes)
            addv = lax.shift_left(jnp.int32(1), (v & 7) * 4)
            idx_all[pl.ds(o, 16)] = jnp.where(m, w, 0)
            val_all[pl.ds(o, 16)] = jnp.where(m, addv, 0)
        return 0

    lax.fori_loop(0, nch, scomp, 0)
    pltpu.sync_copy(val_all, cnt_tab.at[idx_all], add=True)
    plsc.subcore_barrier()

    def gidx_comp(src_ref):
        def body(j, _):
            for kk in range(8):
                o = j * CHUNK + kk * 16
                v = src_ref[pl.ds(o, 16)]
                m, w = own_word(v, tile_base + o + lanes)
                idx_all[pl.ds(o, 16)] = jnp.where(m, w, 0)
            return 0
        lax.fori_loop(0, nch, body, 0)

    def extract(src_ref, save_cnts):
        def body(j, _):
            for kk in range(8):
                o = j * CHUNK + kk * 16
                v = src_ref[pl.ds(o, 16)]
                m, _w = own_word(v, tile_base + o + lanes)
                g = gbuf_all[pl.ds(o, 16)]
                cnt = lax.shift_right_logical(g, (v & 7) * 4) & 15
                cnt = jnp.where(m, cnt, 0)
                outp[pl.ds(o, 16)] = cnt
                if save_cnts:
                    cnts_all[pl.ds(o, 16)] = cnt
            return 0
        lax.fori_loop(0, nch, body, 0)

    gidx_comp(codes_all)
    pltpu.sync_copy(cnt_tab.at[idx_all], gbuf_all)
    extract(codes_all, True)
    pltpu.sync_copy(outp, cntc_hbm.at[c].at[pl.ds(tile_base, PAIRS_PER_TILE)])
    gidx_comp(rcodes_all)
    pltpu.sync_copy(cnt_tab.at[idx_all], gbuf_all)
    extract(rcodes_all, False)
    pltpu.sync_copy(outp, cntr_hbm.at[c].at[pl.ds(tile_base, PAIRS_PER_TILE)])

    def dcomp(j, _):
        for kk in range(8):
            o = j * CHUNK + kk * 16
            v = codes_all[pl.ds(o, 16)]
            gidx = tile_base + o + lanes
            m, _w = own_word(v, gidx)
            m = m & (cnts_all[pl.ds(o, 16)] >= 2)
            slot = lax.shift_right_logical(v * HASH_C, 16) & 0xFFFF
            idx_all[pl.ds(o, 16)] = jnp.where(m, slot, 0)
            val_all[pl.ds(o, 16)] = jnp.where(m, gidx, 0)
        return 0

    lax.fori_loop(0, nch, dcomp, 0)
    pltpu.sync_copy(val_all, dup_tab.at[idx_all], add=True)
    plsc.subcore_barrier()

    pltpu.sync_copy(dup_tab.at[idx_all], gbuf_all)

    def ecomp(j, _):
        for kk in range(8):
            o = j * CHUNK + kk * 16
            v = codes_all[pl.ds(o, 16)]
            m, _w = own_word(v, tile_base + o + lanes)
            m = m & (cnts_all[pl.ds(o, 16)] >= 2)
            outp[pl.ds(o, 16)] = jnp.where(m, gbuf_all[pl.ds(o, 16)], 0)
        return 0

    lax.fori_loop(0, nch, ecomp, 0)
    pltpu.sync_copy(outp, dsum_hbm.at[c].at[pl.ds(tile_base, PAIRS_PER_TILE)])


def _k2(codes, rcodes):
    fn = pl.kernel(
        _k2_body,
        mesh=plsc.VectorSubcoreMesh(**SC_MESH),
        out_type=[
            jax.ShapeDtypeStruct((2, NP), jnp.int32),
            jax.ShapeDtypeStruct((2, NP), jnp.int32),
            jax.ShapeDtypeStruct((2, NP), jnp.int32),
        ],
        scratch_types=[
            pltpu.VMEM((PAIRS_PER_TILE,), jnp.int32),
            pltpu.VMEM((PAIRS_PER_TILE,), jnp.int32),
            pltpu.VMEM((PAIRS_PER_TILE,), jnp.int32),
            pltpu.VMEM((PAIRS_PER_TILE,), jnp.int32),
            pltpu.VMEM((PAIRS_PER_TILE,), jnp.int32),
            pltpu.VMEM((PAIRS_PER_TILE,), jnp.int32),
            pltpu.VMEM((PAIRS_PER_TILE,), jnp.int32),
            pltpu.VMEM((16384,), jnp.int32),
            pltpu.SemaphoreType.DMA,
            pltpu.VMEM_SHARED((1 << 20,), jnp.int32),
            pltpu.VMEM_SHARED((1 << 16,), jnp.int32),
        ],
    )
    return fn(codes, rcodes)


def _k3b_body(adj_ref, out_ref):
    out_ref[...] = jnp.sum(adj_ref[...], axis=1).reshape(1, 1, 128)


def _k3b(adj):
    return pl.pallas_call(
        _k3b_body,
        grid=(32,),
        in_specs=[pl.BlockSpec((128, N), lambda i: (i, 0))],
        out_specs=pl.BlockSpec((1, 1, 128), lambda i: (i, 0, 0)),
        out_shape=jax.ShapeDtypeStruct((32, 1, 128), jnp.float32),
    )(adj)


def _k3a_body(act_ref, code_ref, rcode_ref, cc0_ref, cc1_ref, cr0_ref,
              cr1_ref, ds0_ref, ds1_ref, packed_ref, dest_ref, nn_ref):
    act = act_ref[...]
    cnt = cc0_ref[...] + cc1_ref[...]
    cntr = cr0_ref[...] + cr1_ref[...]
    dsum = ds0_ref[...] + ds1_ref[...]
    ri = lax.broadcasted_iota(jnp.int32, (NROWS, 128), 0)
    ci = lax.broadcasted_iota(jnp.int32, (NROWS, 128), 1)
    gi = ri * 128 + ci
    surv_a = (cnt == 1) | ((cnt == 2) & (2 * gi > dsum))
    cand = act != 0
    tag = lax.shift_left((act > 0).astype(jnp.int32), 24) + jnp.int32(1 << 25)
    surv0 = cand & surv_a & (cntr == 0)
    surv1 = cand & surv_a
    packed_ref[0:NROWS, :] = jnp.where(surv0, code_ref[...] + tag, 0)
    packed_ref[NROWS:2 * NROWS, :] = jnp.where(surv1, rcode_ref[...] + tag, 0)

    tri = (lax.broadcasted_iota(jnp.int32, (NROWS, NROWS), 0)
           > lax.broadcasted_iota(jnp.int32, (NROWS, NROWS), 1)
           ).astype(jnp.float32)
    lt = (lax.broadcasted_iota(jnp.int32, (128, 128), 0)
          <= lax.broadcasted_iota(jnp.int32, (128, 128), 1)
          ).astype(jnp.float32)

    def prefix(sv):
        v = sv.astype(jnp.float32)
        incl = jnp.dot(v, lt, preferred_element_type=jnp.float32)
        rowtot = incl[:, 127:128]
        row_off = jnp.dot(tri, rowtot, preferred_element_type=jnp.float32)
        dest = (row_off + incl - v).astype(jnp.int32)
        n = jnp.sum(sv.astype(jnp.int32))
        return dest, n

    d0, n0 = prefix(surv0)
    d1, n1 = prefix(surv1)
    dest_ref[0:NROWS, :] = d0
    dest_ref[NROWS:2 * NROWS, :] = d1
    nn_ref[0, 0] = n0
    nn_ref[1, 0] = n1


def _k3a(act, code, rcode, cc0, cc1, cr0, cr1, ds0, ds1):
    return pl.pallas_call(
        _k3a_body,
        in_specs=[pl.BlockSpec(memory_space=pltpu.VMEM)] * 9,
        out_specs=[
            pl.BlockSpec(memory_space=pltpu.VMEM),
            pl.BlockSpec(memory_space=pltpu.VMEM),
            pl.BlockSpec(memory_space=pltpu.SMEM),
        ],
        out_shape=[
            jax.ShapeDtypeStruct((2 * NROWS, 128), jnp.int32),
            jax.ShapeDtypeStruct((2 * NROWS, 128), jnp.int32),
            jax.ShapeDtypeStruct((16, 1), jnp.int32),
        ],
    )(act, code, rcode, cc0, cc1, cr0, cr1, ds0, ds1)


def _k4_body(packed_hbm, dest_hbm, nn_hbm, adj_hbm,
             ovrow_hbm, ovcol_hbm, ovdelta_hbm, degd_hbm,
             pall, dall, idx_all, val_all, pvec, idx64, rowb, colb, deltab,
             oldb, nbuf, zbi, zbf, sem, ov_sp, degd_sp):
    c = lax.axis_index("c")
    s = lax.axis_index("s")
    lanes = jnp.arange(16, dtype=jnp.int32)
    tile_base = s * PAIRS_PER_TILE

    for i in range(16):
        zbi[pl.ds(i * 16, 16)] = jnp.zeros((16,), jnp.int32)
        zbf[pl.ds(i * 16, 16)] = jnp.zeros((16,), jnp.float32)
    h_p = pltpu.async_copy(
        packed_hbm.at[c].at[pl.ds(tile_base, PAIRS_PER_TILE)], pall, sem)
    h_d = pltpu.async_copy(
        dest_hbm.at[c].at[pl.ds(tile_base, PAIRS_PER_TILE)], dall, sem)
    pltpu.sync_copy(zbi.at[pl.ds(0, 64)], ov_sp.at[pl.ds(s * 64, 64)])
    pltpu.sync_copy(zbf, degd_sp.at[pl.ds(s * 256, 256)])
    pltpu.sync_copy(nn_hbm, nbuf)
    h_p.wait()
    h_d.wait()
    plsc.subcore_barrier()

    def pcomp(j, _):
        for kk in range(8):
            o = j * CHUNK + kk * 16
            v = pall[pl.ds(o, 16)]
            dd = dall[pl.ds(o, 16)]
            m = (lax.shift_right_logical(v, 25) & 1) == 1
            idx_all[pl.ds(o, 16)] = jnp.where(m, dd, 0)
            val_all[pl.ds(o, 16)] = jnp.where(m, v, 0)
        return 0

    lax.fori_loop(0, CHUNKS_PER_TILE, pcomp, 0)
    pltpu.sync_copy(val_all, ov_sp.at[idx_all], add=True)
    plsc.subcore_barrier()

    slot_base = s * 64
    pltpu.sync_copy(ov_sp.at[pl.ds(slot_base, 64)], pvec.at[pl.ds(0, 64)])
    nv = nbuf[pl.ds(0, 16)]
    n_c = jnp.where(c == 0, nv[0], nv[1])
    for kk in range(4):
        v = pvec[pl.ds(kk * 16, 16)]
        live = (slot_base + kk * 16 + lanes) < n_c
        idx64[pl.ds(kk * 16, 16)] = jnp.where(live, v & 0xFFFFFF, 0)
    pltpu.sync_copy(adj_hbm.at[idx64], oldb)
    for kk in range(4):
        v = pvec[pl.ds(kk * 16, 16)]
        live = (slot_base + kk * 16 + lanes) < n_c
        codev = v & 0xFFFFFF
        newv = (lax.shift_right_logical(v, 24) & 1).astype(jnp.float32)
        delta = jnp.where(live, newv - oldb[pl.ds(kk * 16, 16)], 0.0)
        rowb[pl.ds(kk * 16, 16)] = jnp.where(
            live, lax.shift_right_logical(codev, 12), 0)
        colb[pl.ds(kk * 16, 16)] = jnp.where(live, codev & 4095, 0)
        deltab[pl.ds(kk * 16, 16)] = delta
    pltpu.sync_copy(rowb, ovrow_hbm.at[c].at[pl.ds(slot_base, 64)])
    pltpu.sync_copy(colb, ovcol_hbm.at[c].at[pl.ds(slot_base, 64)])
    pltpu.sync_copy(deltab, ovdelta_hbm.at[c].at[pl.ds(slot_base, 64)])
    pltpu.sync_copy(deltab, degd_sp.at[rowb], add=True)
    plsc.subcore_barrier()

    pltpu.sync_copy(degd_sp.at[pl.ds(s * 256, 256)],
                    degd_hbm.at[c].at[pl.ds(s * 256, 256)])


def _k4(packed, dest, nn, adj_flat):
    fn = pl.kernel(
        _k4_body,
        mesh=plsc.VectorSubcoreMesh(**SC_MESH),
        out_type=[
            jax.ShapeDtypeStruct((2, MAXOV), jnp.int32),
            jax.ShapeDtypeStruct((2, MAXOV), jnp.int32),
            jax.ShapeDtypeStruct((2, MAXOV), jnp.float32),
            jax.ShapeDtypeStruct((2, N), jnp.float32),
        ],
        scratch_types=[
            pltpu.VMEM((PAIRS_PER_TILE,), jnp.int32),
            pltpu.VMEM((PAIRS_PER_TILE,), jnp.int32),
            pltpu.VMEM((PAIRS_PER_TILE,), jnp.int32),
            pltpu.VMEM((PAIRS_PER_TILE,), jnp.int32),
            pltpu.VMEM((CHUNK,), jnp.int32),
            pltpu.VMEM((64,), jnp.int32),
            pltpu.VMEM((64,), jnp.int32),
            pltpu.VMEM((64,), jnp.int32),
            pltpu.VMEM((64,), jnp.float32),
            pltpu.VMEM((64,), jnp.float32),
            pltpu.VMEM((16,), jnp.int32),
            pltpu.VMEM((256,), jnp.int32),
            pltpu.VMEM((256,), jnp.float32),
            pltpu.SemaphoreType.DMA,
            pltpu.VMEM_SHARED((MAXOV,), jnp.int32),
            pltpu.VMEM_SHARED((N,), jnp.float32),
        ],
    )
    return fn(packed, dest, nn, adj_flat)


def _k5_body(bd_ref, dd0_ref, dd1_ref, x1_ref, d_ref, xs1_ref):
    deg = bd_ref[...] + dd0_ref[...] + dd1_ref[...] + 1.0
    d = lax.rsqrt(jnp.maximum(deg, 1e-12))
    d_ref[...] = d
    xs1_ref[:, 0:NHID] = d * x1_ref[...]
    xs1_ref[:, NHID:128] = jnp.zeros((N, 128 - NHID), jnp.float32)


def _k5(base_deg, dd0, dd1, x1):
    return pl.pallas_call(
        _k5_body,
        in_specs=[pl.BlockSpec(memory_space=pltpu.VMEM)] * 4,
        out_specs=[pl.BlockSpec(memory_space=pltpu.VMEM)] * 2,
        out_shape=[
            jax.ShapeDtypeStruct((N, 1), jnp.float32),
            jax.ShapeDtypeStruct((N, 128), jnp.float32),
        ],
    )(base_deg, dd0, dd1, x1)


def _urow_body(width, ovrow_hbm, ovcol_hbm, ovdelta_hbm, tab_hbm, out_hbm,
               rowb, colb, deltab, rows_v, zrow, u_sp):
    c = lax.axis_index("c")
    s = lax.axis_index("s")
    rows_per_tile = N // NSUB

    for i in range(16):
        for kk in range(width // 16):
            zrow[i, pl.ds(kk * 16, 16)] = jnp.zeros((16,), jnp.float32)

    def zstep(j, _):
        pltpu.sync_copy(zrow, u_sp.at[pl.ds(s * rows_per_tile + j * 16, 16)])
        return 0

    lax.fori_loop(0, rows_per_tile // 16, zstep, 0)
    base = s * 64
    pltpu.sync_copy(ovrow_hbm.at[c].at[pl.ds(base, 64)], rowb)
    pltpu.sync_copy(ovcol_hbm.at[c].at[pl.ds(base, 64)], colb)
    pltpu.sync_copy(ovdelta_hbm.at[c].at[pl.ds(base, 64)], deltab)
    plsc.subcore_barrier()

    pltpu.sync_copy(tab_hbm.at[colb], rows_v)
    dvs = [deltab[pl.ds(q * 16, 16)] for q in range(4)]
    for i in range(64):
        dsc = dvs[i // 16][i % 16]
        for kk in range(width // 16):
            rows_v[i, pl.ds(kk * 16, 16)] = rows_v[i, pl.ds(kk * 16, 16)] * dsc
    pltpu.sync_copy(rows_v, u_sp.at[rowb], add=True)
    plsc.subcore_barrier()

    pltpu.sync_copy(u_sp.at[pl.ds(s * rows_per_tile, rows_per_tile)],
                    out_hbm.at[c].at[pl.ds(s * rows_per_tile, rows_per_tile)])


def _k_urow(width, ovrow, ovcol, ovdelta, tab):
    def body(*refs):
        _urow_body(width, *refs)

    fn = pl.kernel(
        body,
        mesh=plsc.VectorSubcoreMesh(**SC_MESH),
        out_type=[jax.ShapeDtypeStruct((2, N, width), jnp.float32)],
        scratch_types=[
            pltpu.VMEM((64,), jnp.int32),
            pltpu.VMEM((64,), jnp.int32),
            pltpu.VMEM((64,), jnp.float32),
            pltpu.VMEM((64, width), jnp.float32),
            pltpu.VMEM((16, width), jnp.float32),
            pltpu.VMEM_SHARED((N, width), jnp.float32),
        ],
    )
    return fn(ovrow, ovcol, ovdelta, tab)[0]


def _k7_body(adj_ref, xs1_ref, u1a_ref, u1b_ref, d_ref, b1_ref, w2_ref,
             gs_ref):
    i = pl.program_id(0)
    y = jnp.dot(adj_ref[...], xs1_ref[...], preferred_element_type=jnp.float32)
    rows = xs1_ref[pl.ds(i * 256, 256), :]
    t = d_ref[...] * (y + u1a_ref[...] + u1b_ref[...] + rows) + b1_ref[...]
    h = jnp.maximum(t, 0.0)
    gs_ref[...] = d_ref[...] * jnp.dot(h, w2_ref[...],
                                       preferred_element_type=jnp.float32)


def _k7(adj, xs1, u1a, u1b, d, b1, w2p):
    return pl.pallas_call(
        _k7_body,
        grid=(16,),
        in_specs=[
            pl.BlockSpec((256, N), lambda i: (i, 0)),
            pl.BlockSpec((N, 128), lambda i: (0, 0)),
            pl.BlockSpec((256, 128), lambda i: (i, 0)),
            pl.BlockSpec((256, 128), lambda i: (i, 0)),
            pl.BlockSpec((256, 1), lambda i: (i, 0)),
            pl.BlockSpec((1, 128), lambda i: (0, 0)),
            pl.BlockSpec((128, 128), lambda i: (0, 0)),
        ],
        out_specs=pl.BlockSpec((256, 128), lambda i: (i, 0)),
        out_shape=jax.ShapeDtypeStruct((N, 128), jnp.float32),
    )(adj, xs1, u1a, u1b, d, b1, w2p)


def _k9_body(adj_ref, gs_ref, u2a_ref, u2b_ref, d_ref, b2_ref, out_ref):
    i = pl.program_id(0)
    y = jnp.dot(adj_ref[...], gs_ref[...], preferred_element_type=jnp.float32)
    rows = gs_ref[pl.ds(i * 256, 256), :]
    out_ref[...] = (d_ref[...] * (y + u2a_ref[...] + u2b_ref[...] + rows)
                    + b2_ref[...])


def _k9(adj, gs, u2a, u2b, d, b2p):
    return pl.pallas_call(
        _k9_body,
        grid=(16,),
        in_specs=[
            pl.BlockSpec((256, N), lambda i: (i, 0)),
            pl.BlockSpec((N, 128), lambda i: (0, 0)),
            pl.BlockSpec((256, 128), lambda i: (i, 0)),
            pl.BlockSpec((256, 128), lambda i: (i, 0)),
            pl.BlockSpec((256, 1), lambda i: (i, 0)),
            pl.BlockSpec((1, 128), lambda i: (0, 0)),
        ],
        out_specs=pl.BlockSpec((256, 128), lambda i: (i, 0)),
        out_shape=jax.ShapeDtypeStruct((N, 128), jnp.float32),
    )(adj, gs, u2a, u2b, d, b2p)


def kernel(extended_sub_adj, sub_feat, M, pairs, W1, b1, W2, b2, top_k):
    pad = NP - NMASK
    m_p = jnp.pad(M, (0, pad)).reshape(NROWS, 128)
    src = jnp.pad(pairs[:, 0], (0, pad)).reshape(NROWS, 128)
    dst = jnp.pad(pairs[:, 1], (0, pad)).reshape(NROWS, 128)
    tk = jnp.asarray(top_k, jnp.int32).reshape(1, 1)

    act, code, rcode, x1 = _k1(m_p, src, dst, sub_feat, W1, tk)
    cntc, cntr, dsum = _k2(code.reshape(NP), rcode.reshape(NP))
    base_deg = _k3b(extended_sub_adj).reshape(N, 1)
    packed, dest, nn = _k3a(
        act, code, rcode,
        cntc[0].reshape(NROWS, 128), cntc[1].reshape(NROWS, 128),
        cntr[0].reshape(NROWS, 128), cntr[1].reshape(NROWS, 128),
        dsum[0].reshape(NROWS, 128), dsum[1].reshape(NROWS, 128))
    ovrow, ovcol, ovdelta, degd = _k4(
        packed.reshape(2, NP), dest.reshape(2, NP), nn.reshape(16),
        extended_sub_adj.reshape(N * N))
    d, xs1 = _k5(base_deg, degd[0].reshape(N, 1), degd[1].reshape(N, 1), x1)
    u1 = _k_urow(128, ovrow, ovcol, ovdelta, xs1)
    w2p = jnp.pad(W2, ((0, 128 - NHID), (0, 128 - NCLASS)))
    b1p = jnp.pad(b1, (0, 128 - NHID)).reshape(1, 128)
    b2p = jnp.pad(b2, (0, 128 - NCLASS)).reshape(1, 128)
    gs = _k7(extended_sub_adj, xs1, u1[0], u1[1], d, b1p, w2p)
    u2 = _k_urow(128, ovrow, ovcol, ovdelta, gs)
    out = _k9(extended_sub_adj, gs, u2[0], u2[1], d, b2p)
    return out[:, :NCLASS]


# speedup vs baseline: 3.6495x; 1.0458x over previous
import jax
import jax.numpy as jnp
from jax import lax
from jax.experimental import pallas as pl
from jax.experimental.pallas import tpu as pltpu
from jax.experimental.pallas import tpu_sc as plsc

N = 4096
NHID = 64
NCLASS = 8
NMASK = 100000
TOPK = 1024
NP = 100352
NROWS = NP // 128
ONE_BITS = 0x3F800000
MAXOV = 1024
HALF_CODES = 1 << 23
HASH_C = -1640531527

NSUB = 16
PAIRS_PER_TILE = NP // NSUB
CHUNK = 128
CHUNKS_PER_TILE = PAIRS_PER_TILE // CHUNK
SC_MESH = dict(core_axis_name="c", subcore_axis_name="s")


def _k1_body(m_ref, src_ref, dst_ref, feat_ref, w1_ref, tk_ref,
             act_ref, code_ref, rcode_ref, x1_ref):
    m = m_ref[...]
    tm = jnp.tanh(m)
    a = jnp.abs(tm)
    ri = lax.broadcasted_iota(jnp.int32, (NROWS, 128), 0)
    ci = lax.broadcasted_iota(jnp.int32, (NROWS, 128), 1)
    gi = ri * 128 + ci
    valid = gi < NMASK
    key = jnp.where(valid, lax.bitcast_convert_type(a, jnp.int32),
                    jnp.int32(-1))
    k = jnp.clip(tk_ref[0, 0], 0, TOPK)

    def vstep(_, lohi):
        lo, hi = lohi
        mid = lax.div(lo + hi, 2)
        c = jnp.sum((key > mid).astype(jnp.int32))
        take_hi = c < k
        return (jnp.where(take_hi, lo, mid + 1), jnp.where(take_hi, mid, hi))

    thr, _ = lax.fori_loop(0, 31, vstep, (jnp.int32(0), jnp.int32(ONE_BITS)))
    eq = key == thr
    n_gt = jnp.sum((key > thr).astype(jnp.int32))
    tn = k - n_gt

    def istep(_, lohi):
        lo2, hi2 = lohi
        mid = lax.div(lo2 + hi2, 2)
        h = jnp.sum((eq & (gi < mid)).astype(jnp.int32))
        ge = h >= tn
        return (jnp.where(ge, lo2, mid + 1), jnp.where(ge, mid, hi2))

    cut, _ = lax.fori_loop(0, 17, istep, (jnp.int32(0), jnp.int32(1 << 17)))
    keep = (key > thr) | (eq & (gi < cut))
    act_ref[...] = jnp.where(
        keep & (tm > 0.5), jnp.int32(1),
        jnp.where(keep & (tm < -0.5), jnp.int32(-1), jnp.int32(0)))
    s = src_ref[...]
    d = dst_ref[...]
    code_ref[...] = lax.shift_left(s, 12) + d
    rcode_ref[...] = lax.shift_left(d, 12) + s
    x1_ref[...] = jnp.dot(feat_ref[...], w1_ref[...],
                          preferred_element_type=jnp.float32)


def _k1(m_p, src_p, dst_p, sub_feat, w1, tk):
    return pl.pallas_call(
        _k1_body,
        in_specs=[pl.BlockSpec(memory_space=pltpu.VMEM)] * 5
        + [pl.BlockSpec(memory_space=pltpu.SMEM)],
        out_specs=[pl.BlockSpec(memory_space=pltpu.VMEM)] * 4,
        out_shape=[
            jax.ShapeDtypeStruct((NROWS, 128), jnp.int32),
            jax.ShapeDtypeStruct((NROWS, 128), jnp.int32),
            jax.ShapeDtypeStruct((NROWS, 128), jnp.int32),
            jax.ShapeDtypeStruct((N, NHID), jnp.float32),
        ],
    )(m_p, src_p, dst_p, sub_feat, w1, tk)


def _k2_body(codes_hbm, rcodes_hbm,
             cntc_hbm, cntr_hbm, dsum_hbm,
             codes_all, rcodes_all, idx_all, val_all, gbuf_all,
             cnts_all, outp, idxd, vald, zbuf, sem,
             cnt_tab, dup_tab):
    c = lax.axis_index("c")
    s = lax.axis_index("s")
    lanes = jnp.arange(16, dtype=jnp.int32)
    tile_base = s * PAIRS_PER_TILE
    nch = CHUNKS_PER_TILE

    def own_word(v, gidx):
        m = (gidx < NMASK) & (lax.shift_right_logical(v, 23) == c)
        w = lax.shift_right_logical(v & (HALF_CODES - 1), 4)
        return m, w


    h_c = pltpu.async_copy(
        codes_hbm.at[pl.ds(tile_base, PAIRS_PER_TILE)], codes_all, sem)
    h_r = pltpu.async_copy(
        rcodes_hbm.at[pl.ds(tile_base, PAIRS_PER_TILE)], rcodes_all, sem)

    def zfill(j, _):
        zbuf[pl.ds(j * 16, 16)] = jnp.zeros((16,), jnp.int32)
        return 0

    lax.fori_loop(0, 1024, zfill, 0)
    for j in range(2):
        pltpu.sync_copy(zbuf, cnt_tab.at[pl.ds(s * 32768 + j * 16384, 16384)])
    pltpu.sync_copy(zbuf.at[pl.ds(0, 4096)], dup_tab.at[pl.ds(s * 4096, 4096)])
    h_c.wait()
    h_r.wait()
    plsc.subcore_barrier()

    @plsc.parallel_loop(0, nch, unroll=4)
    def _s(j):
        for kk in range(8):
            o = j * CHUNK + kk * 16
            v = codes_all[pl.ds(o, 16)]
            m, w = own_word(v, tile_base + o + lanes)
            addv = lax.shift_left(jnp.int32(1), (v & 15) * 2)
            idx_all[pl.ds(o, 16)] = jnp.where(m, w, 0)
            val_all[pl.ds(o, 16)] = jnp.where(m, addv, 0)

    pltpu.sync_copy(val_all, cnt_tab.at[idx_all], add=True)
    plsc.subcore_barrier()

    pltpu.sync_copy(cnt_tab.at[idx_all], gbuf_all)

    @plsc.parallel_loop(0, nch, unroll=4)
    def _m(j):
        for kk in range(8):
            o = j * CHUNK + kk * 16
            v = codes_all[pl.ds(o, 16)]
            gidx = tile_base + o + lanes
            m, _w = own_word(v, gidx)
            g = gbuf_all[pl.ds(o, 16)]
            cnt = lax.shift_right_logical(g, (v & 15) * 2) & 3
            cnt = jnp.where(m, cnt, 0)
            outp[pl.ds(o, 16)] = cnt
            cnts_all[pl.ds(o, 16)] = cnt
            mdup = m & (cnt >= 2)
            slot = lax.shift_right_logical(v * HASH_C, 16) & 0xFFFF
            idxd[pl.ds(o, 16)] = jnp.where(mdup, slot, 0)
            vald[pl.ds(o, 16)] = jnp.where(mdup, gidx, 0)

    pltpu.sync_copy(outp, cntc_hbm.at[c].at[pl.ds(tile_base, PAIRS_PER_TILE)])
    pltpu.sync_copy(vald, dup_tab.at[idxd], add=True)

    @plsc.parallel_loop(0, nch, unroll=4)
    def _gr(j):
        for kk in range(8):
            o = j * CHUNK + kk * 16
            v = rcodes_all[pl.ds(o, 16)]
            m, w = own_word(v, tile_base + o + lanes)
            idx_all[pl.ds(o, 16)] = jnp.where(m, w, 0)

    pltpu.sync_copy(cnt_tab.at[idx_all], gbuf_all)

    @plsc.parallel_loop(0, nch, unroll=4)
    def _xr(j):
        for kk in range(8):
            o = j * CHUNK + kk * 16
            v = rcodes_all[pl.ds(o, 16)]
            m, _w = own_word(v, tile_base + o + lanes)
            g = gbuf_all[pl.ds(o, 16)]
            cnt = lax.shift_right_logical(g, (v & 15) * 2) & 3
            outp[pl.ds(o, 16)] = jnp.where(m, cnt, 0)

    pltpu.sync_copy(outp, cntr_hbm.at[c].at[pl.ds(tile_base, PAIRS_PER_TILE)])
    plsc.subcore_barrier()

    pltpu.sync_copy(dup_tab.at[idxd], gbuf_all)

    @plsc.parallel_loop(0, nch, unroll=4)
    def _e(j):
        for kk in range(8):
            o = j * CHUNK + kk * 16
            v = codes_all[pl.ds(o, 16)]
            m, _w = own_word(v, tile_base + o + lanes)
            m = m & (cnts_all[pl.ds(o, 16)] >= 2)
            outp[pl.ds(o, 16)] = jnp.where(m, gbuf_all[pl.ds(o, 16)], 0)

    pltpu.sync_copy(outp, dsum_hbm.at[c].at[pl.ds(tile_base, PAIRS_PER_TILE)])


def _k2(codes, rcodes):
    fn = pl.kernel(
        _k2_body,
        mesh=plsc.VectorSubcoreMesh(**SC_MESH),
        out_type=[
            jax.ShapeDtypeStruct((2, NP), jnp.int32),
            jax.ShapeDtypeStruct((2, NP), jnp.int32),
            jax.ShapeDtypeStruct((2, NP), jnp.int32),
        ],
        scratch_types=[
            pltpu.VMEM((PAIRS_PER_TILE,), jnp.int32),
            pltpu.VMEM((PAIRS_PER_TILE,), jnp.int32),
            pltpu.VMEM((PAIRS_PER_TILE,), jnp.int32),
            pltpu.VMEM((PAIRS_PER_TILE,), jnp.int32),
            pltpu.VMEM((PAIRS_PER_TILE,), jnp.int32),
            pltpu.VMEM((PAIRS_PER_TILE,), jnp.int32),
            pltpu.VMEM((PAIRS_PER_TILE,), jnp.int32),
            pltpu.VMEM((PAIRS_PER_TILE,), jnp.int32),
            pltpu.VMEM((PAIRS_PER_TILE,), jnp.int32),
            pltpu.VMEM((16384,), jnp.int32),
            pltpu.SemaphoreType.DMA,
            pltpu.VMEM_SHARED((1 << 19,), jnp.int32),
            pltpu.VMEM_SHARED((1 << 16,), jnp.int32),
        ],
    )
    return fn(codes, rcodes)


def _k3b_body(adj_ref, out_ref):
    out_ref[...] = jnp.sum(adj_ref[...], axis=1).reshape(1, 1, 128)


def _k3b(adj):
    return pl.pallas_call(
        _k3b_body,
        grid=(32,),
        in_specs=[pl.BlockSpec((128, N), lambda i: (i, 0))],
        out_specs=pl.BlockSpec((1, 1, 128), lambda i: (i, 0, 0)),
        out_shape=jax.ShapeDtypeStruct((32, 1, 128), jnp.float32),
    )(adj)


def _k3a_body(act_ref, code_ref, rcode_ref, cc0_ref, cc1_ref, cr0_ref,
              cr1_ref, ds0_ref, ds1_ref, packed_ref, dest_ref, nn_ref):
    act = act_ref[...]
    cnt = cc0_ref[...] + cc1_ref[...]
    cntr = cr0_ref[...] + cr1_ref[...]
    dsum = ds0_ref[...] + ds1_ref[...]
    ri = lax.broadcasted_iota(jnp.int32, (NROWS, 128), 0)
    ci = lax.broadcasted_iota(jnp.int32, (NROWS, 128), 1)
    gi = ri * 128 + ci
    surv_a = (cnt == 1) | ((cnt == 2) & (2 * gi > dsum))
    cand = act != 0
    tag = lax.shift_left((act > 0).astype(jnp.int32), 24) + jnp.int32(1 << 25)
    surv0 = cand & surv_a & (cntr == 0)
    surv1 = cand & surv_a
    packed_ref[0:NROWS, :] = jnp.where(surv0, code_ref[...] + tag, 0)
    packed_ref[NROWS:2 * NROWS, :] = jnp.where(surv1, rcode_ref[...] + tag, 0)

    tri = (lax.broadcasted_iota(jnp.int32, (NROWS, NROWS), 0)
           > lax.broadcasted_iota(jnp.int32, (NROWS, NROWS), 1)
           ).astype(jnp.float32)
    lt = (lax.broadcasted_iota(jnp.int32, (128, 128), 0)
          <= lax.broadcasted_iota(jnp.int32, (128, 128), 1)
          ).astype(jnp.float32)

    def prefix(sv):
        v = sv.astype(jnp.float32)
        incl = jnp.dot(v, lt, preferred_element_type=jnp.float32)
        rowtot = incl[:, 127:128]
        row_off = jnp.dot(tri, rowtot, preferred_element_type=jnp.float32)
        dest = (row_off + incl - v).astype(jnp.int32)
        n = jnp.sum(sv.astype(jnp.int32))
        return dest, n

    d0, n0 = prefix(surv0)
    d1, n1 = prefix(surv1)
    dest_ref[0:NROWS, :] = d0
    dest_ref[NROWS:2 * NROWS, :] = d1
    nn_ref[0, 0] = n0
    nn_ref[1, 0] = n1


def _k3a(act, code, rcode, cc0, cc1, cr0, cr1, ds0, ds1):
    return pl.pallas_call(
        _k3a_body,
        in_specs=[pl.BlockSpec(memory_space=pltpu.VMEM)] * 9,
        out_specs=[
            pl.BlockSpec(memory_space=pltpu.VMEM),
            pl.BlockSpec(memory_space=pltpu.VMEM),
            pl.BlockSpec(memory_space=pltpu.SMEM),
        ],
        out_shape=[
            jax.ShapeDtypeStruct((2 * NROWS, 128), jnp.int32),
            jax.ShapeDtypeStruct((2 * NROWS, 128), jnp.int32),
            jax.ShapeDtypeStruct((16, 1), jnp.int32),
        ],
    )(act, code, rcode, cc0, cc1, cr0, cr1, ds0, ds1)


def _k4_body(packed_hbm, dest_hbm, nn_hbm, adj_hbm,
             ovrow_hbm, ovcol_hbm, ovdelta_hbm, degd_hbm,
             pall, dall, idx_all, val_all, pvec, idx64, rowb, colb, deltab,
             oldb, nbuf, zbi, zbf, sem, ov_sp, degd_sp):
    c = lax.axis_index("c")
    s = lax.axis_index("s")
    lanes = jnp.arange(16, dtype=jnp.int32)
    tile_base = s * PAIRS_PER_TILE

    for i in range(16):
        zbi[pl.ds(i * 16, 16)] = jnp.zeros((16,), jnp.int32)
        zbf[pl.ds(i * 16, 16)] = jnp.zeros((16,), jnp.float32)
    h_p = pltpu.async_copy(
        packed_hbm.at[c].at[pl.ds(tile_base, PAIRS_PER_TILE)], pall, sem)
    h_d = pltpu.async_copy(
        dest_hbm.at[c].at[pl.ds(tile_base, PAIRS_PER_TILE)], dall, sem)
    pltpu.sync_copy(zbi.at[pl.ds(0, 64)], ov_sp.at[pl.ds(s * 64, 64)])
    pltpu.sync_copy(zbf, degd_sp.at[pl.ds(s * 256, 256)])
    pltpu.sync_copy(nn_hbm, nbuf)
    h_p.wait()
    h_d.wait()
    plsc.subcore_barrier()

    @plsc.parallel_loop(0, CHUNKS_PER_TILE, unroll=4)
    def _p(j):
        for kk in range(8):
            o = j * CHUNK + kk * 16
            v = pall[pl.ds(o, 16)]
            dd = dall[pl.ds(o, 16)]
            m = (lax.shift_right_logical(v, 25) & 1) == 1
            idx_all[pl.ds(o, 16)] = jnp.where(m, dd, 0)
            val_all[pl.ds(o, 16)] = jnp.where(m, v, 0)
    pltpu.sync_copy(val_all, ov_sp.at[idx_all], add=True)
    plsc.subcore_barrier()

    slot_base = s * 64
    pltpu.sync_copy(ov_sp.at[pl.ds(slot_base, 64)], pvec.at[pl.ds(0, 64)])
    nv = nbuf[pl.ds(0, 16)]
    n_c = jnp.where(c == 0, nv[0], nv[1])
    for kk in range(4):
        v = pvec[pl.ds(kk * 16, 16)]
        live = (slot_base + kk * 16 + lanes) < n_c
        idx64[pl.ds(kk * 16, 16)] = jnp.where(live, v & 0xFFFFFF, 0)
    pltpu.sync_copy(adj_hbm.at[idx64], oldb)
    for kk in range(4):
        v = pvec[pl.ds(kk * 16, 16)]
        live = (slot_base + kk * 16 + lanes) < n_c
        codev = v & 0xFFFFFF
        newv = (lax.shift_right_logical(v, 24) & 1).astype(jnp.float32)
        delta = jnp.where(live, newv - oldb[pl.ds(kk * 16, 16)], 0.0)
        rowb[pl.ds(kk * 16, 16)] = jnp.where(
            live, lax.shift_right_logical(codev, 12), 0)
        colb[pl.ds(kk * 16, 16)] = jnp.where(live, codev & 4095, 0)
        deltab[pl.ds(kk * 16, 16)] = delta
    pltpu.sync_copy(rowb, ovrow_hbm.at[c].at[pl.ds(slot_base, 64)])
    pltpu.sync_copy(colb, ovcol_hbm.at[c].at[pl.ds(slot_base, 64)])
    pltpu.sync_copy(deltab, ovdelta_hbm.at[c].at[pl.ds(slot_base, 64)])
    pltpu.sync_copy(deltab, degd_sp.at[rowb], add=True)
    plsc.subcore_barrier()

    pltpu.sync_copy(degd_sp.at[pl.ds(s * 256, 256)],
                    degd_hbm.at[c].at[pl.ds(s * 256, 256)])


def _k4(packed, dest, nn, adj_flat):
    fn = pl.kernel(
        _k4_body,
        mesh=plsc.VectorSubcoreMesh(**SC_MESH),
        out_type=[
            jax.ShapeDtypeStruct((2, MAXOV), jnp.int32),
            jax.ShapeDtypeStruct((2, MAXOV), jnp.int32),
            jax.ShapeDtypeStruct((2, MAXOV), jnp.float32),
            jax.ShapeDtypeStruct((2, N), jnp.float32),
        ],
        scratch_types=[
            pltpu.VMEM((PAIRS_PER_TILE,), jnp.int32),
            pltpu.VMEM((PAIRS_PER_TILE,), jnp.int32),
            pltpu.VMEM((PAIRS_PER_TILE,), jnp.int32),
            pltpu.VMEM((PAIRS_PER_TILE,), jnp.int32),
            pltpu.VMEM((CHUNK,), jnp.int32),
            pltpu.VMEM((64,), jnp.int32),
            pltpu.VMEM((64,), jnp.int32),
            pltpu.VMEM((64,), jnp.int32),
            pltpu.VMEM((64,), jnp.float32),
            pltpu.VMEM((64,), jnp.float32),
            pltpu.VMEM((16,), jnp.int32),
            pltpu.VMEM((256,), jnp.int32),
            pltpu.VMEM((256,), jnp.float32),
            pltpu.SemaphoreType.DMA,
            pltpu.VMEM_SHARED((MAXOV,), jnp.int32),
            pltpu.VMEM_SHARED((N,), jnp.float32),
        ],
    )
    return fn(packed, dest, nn, adj_flat)


def _k5_body(bd_ref, dd0_ref, dd1_ref, x1_ref, d_ref, xs1_ref):
    deg = bd_ref[...] + dd0_ref[...] + dd1_ref[...] + 1.0
    d = lax.rsqrt(jnp.maximum(deg, 1e-12))
    d_ref[...] = d
    xs1_ref[:, 0:NHID] = d * x1_ref[...]
    xs1_ref[:, NHID:128] = jnp.zeros((N, 128 - NHID), jnp.float32)


def _k5(base_deg, dd0, dd1, x1):
    return pl.pallas_call(
        _k5_body,
        in_specs=[pl.BlockSpec(memory_space=pltpu.VMEM)] * 4,
        out_specs=[pl.BlockSpec(memory_space=pltpu.VMEM)] * 2,
        out_shape=[
            jax.ShapeDtypeStruct((N, 1), jnp.float32),
            jax.ShapeDtypeStruct((N, 128), jnp.float32),
        ],
    )(base_deg, dd0, dd1, x1)


def _urow_body(width, ovrow_hbm, ovcol_hbm, ovdelta_hbm, tab_hbm, out_hbm,
               rowb, colb, deltab, idxb, rows_v, zrow, sem, u_sp):
    c = lax.axis_index("c")
    s = lax.axis_index("s")
    lanes = jnp.arange(16, dtype=jnp.int32)
    ebase = s * 128
    row_lo = c * (N // 2)

    h_r = pltpu.async_copy(ovrow_hbm.at[pl.ds(ebase, 128)], rowb, sem)
    h_g = pltpu.async_copy(ovcol_hbm.at[pl.ds(ebase, 128)], colb, sem)
    h_d = pltpu.async_copy(ovdelta_hbm.at[pl.ds(ebase, 128)], deltab, sem)
    for i in range(16):
        for kk in range(width // 16):
            zrow[i, pl.ds(kk * 16, 16)] = jnp.zeros((16,), jnp.float32)

    @plsc.parallel_loop(0, (N // 2 // NSUB) // 16, unroll=2)
    def _z(j):
        pltpu.sync_copy(zrow, u_sp.at[pl.ds(s * (N // 2 // NSUB) + j * 16, 16)])

    h_r.wait()
    h_g.wait()
    h_d.wait()
    plsc.subcore_barrier()

    pltpu.sync_copy(tab_hbm.at[colb], rows_v)
    dm_all = []
    for q in range(8):
        rv = rowb[pl.ds(q * 16, 16)]
        m = lax.shift_right_logical(rv, 11) == c
        idxb[pl.ds(q * 16, 16)] = jnp.where(m, rv - row_lo, 0)
        dm_all.append(jnp.where(m, deltab[pl.ds(q * 16, 16)], 0.0))
    for i in range(128):
        dsc = dm_all[i // 16][i % 16]
        for kk in range(width // 16):
            rows_v[i, pl.ds(kk * 16, 16)] = rows_v[i, pl.ds(kk * 16, 16)] * dsc
    pltpu.sync_copy(rows_v, u_sp.at[idxb], add=True)
    plsc.subcore_barrier()

    rpt = N // 2 // NSUB
    pltpu.sync_copy(u_sp.at[pl.ds(s * rpt, rpt)],
                    out_hbm.at[c].at[pl.ds(s * rpt, rpt)])


def _k_urow(width, ovrow, ovcol, ovdelta, tab):
    def body(*refs):
        _urow_body(width, *refs)

    fn = pl.kernel(
        body,
        mesh=plsc.VectorSubcoreMesh(**SC_MESH),
        out_type=[jax.ShapeDtypeStruct((2, N // 2, width), jnp.float32)],
        scratch_types=[
            pltpu.VMEM((128,), jnp.int32),
            pltpu.VMEM((128,), jnp.int32),
            pltpu.VMEM((128,), jnp.float32),
            pltpu.VMEM((128,), jnp.int32),
            pltpu.VMEM((128, width), jnp.float32),
            pltpu.VMEM((16, width), jnp.float32),
            pltpu.SemaphoreType.DMA,
            pltpu.VMEM_SHARED((N // 2, width), jnp.float32),
        ],
    )
    return fn(ovrow, ovcol, ovdelta, tab)[0]


def _k7_body(adj_ref, xs1_ref, u1_ref, d_ref, b1_ref, w2_ref, gs_ref):
    i = pl.program_id(0)
    y = jnp.dot(adj_ref[...], xs1_ref[...], preferred_element_type=jnp.float32)
    rows = xs1_ref[pl.ds(i * 256, 256), :]
    t = d_ref[...] * (y + u1_ref[0] + rows) + b1_ref[...]
    h = jnp.maximum(t, 0.0)
    gs_ref[...] = d_ref[...] * jnp.dot(h, w2_ref[...],
                                       preferred_element_type=jnp.float32)


def _k7(adj, xs1, u1, d, b1, w2p):
    return pl.pallas_call(
        _k7_body,
        grid=(16,),
        in_specs=[
            pl.BlockSpec((256, N), lambda i: (i, 0)),
            pl.BlockSpec((N, 128), lambda i: (0, 0)),
            pl.BlockSpec((1, 256, 128), lambda i: (i // 8, i % 8, 0)),
            pl.BlockSpec((256, 1), lambda i: (i, 0)),
            pl.BlockSpec((1, 128), lambda i: (0, 0)),
            pl.BlockSpec((128, 128), lambda i: (0, 0)),
        ],
        out_specs=pl.BlockSpec((256, 128), lambda i: (i, 0)),
        out_shape=jax.ShapeDtypeStruct((N, 128), jnp.float32),
    )(adj, xs1, u1, d, b1, w2p)


def _k9_body(adj_ref, gs_ref, u2_ref, d_ref, b2_ref, out_ref):
    i = pl.program_id(0)
    y = jnp.dot(adj_ref[...], gs_ref[...], preferred_element_type=jnp.float32)
    rows = gs_ref[pl.ds(i * 256, 256), :]
    out_ref[...] = d_ref[...] * (y + u2_ref[0] + rows) + b2_ref[...]


def _k9(adj, gs, u2, d, b2p):
    return pl.pallas_call(
        _k9_body,
        grid=(16,),
        in_specs=[
            pl.BlockSpec((256, N), lambda i: (i, 0)),
            pl.BlockSpec((N, 128), lambda i: (0, 0)),
            pl.BlockSpec((1, 256, 128), lambda i: (i // 8, i % 8, 0)),
            pl.BlockSpec((256, 1), lambda i: (i, 0)),
            pl.BlockSpec((1, 128), lambda i: (0, 0)),
        ],
        out_specs=pl.BlockSpec((256, 128), lambda i: (i, 0)),
        out_shape=jax.ShapeDtypeStruct((N, 128), jnp.float32),
    )(adj, gs, u2, d, b2p)


def kernel(extended_sub_adj, sub_feat, M, pairs, W1, b1, W2, b2, top_k):
    pad = NP - NMASK
    m_p = jnp.pad(M, (0, pad)).reshape(NROWS, 128)
    src = jnp.pad(pairs[:, 0], (0, pad)).reshape(NROWS, 128)
    dst = jnp.pad(pairs[:, 1], (0, pad)).reshape(NROWS, 128)
    tk = jnp.asarray(top_k, jnp.int32).reshape(1, 1)

    act, code, rcode, x1 = _k1(m_p, src, dst, sub_feat, W1, tk)
    cntc, cntr, dsum = _k2(code.reshape(NP), rcode.reshape(NP))
    base_deg = _k3b(extended_sub_adj).reshape(N, 1)
    packed, dest, nn = _k3a(
        act, code, rcode,
        cntc[0].reshape(NROWS, 128), cntc[1].reshape(NROWS, 128),
        cntr[0].reshape(NROWS, 128), cntr[1].reshape(NROWS, 128),
        dsum[0].reshape(NROWS, 128), dsum[1].reshape(NROWS, 128))
    ovrow, ovcol, ovdelta, degd = _k4(
        packed.reshape(2, NP), dest.reshape(2, NP), nn.reshape(16),
        extended_sub_adj.reshape(N * N))
    d, xs1 = _k5(base_deg, degd[0].reshape(N, 1), degd[1].reshape(N, 1), x1)
    ovrow_f = ovrow.reshape(2 * MAXOV)
    ovcol_f = ovcol.reshape(2 * MAXOV)
    ovdelta_f = ovdelta.reshape(2 * MAXOV)
    u1 = _k_urow(128, ovrow_f, ovcol_f, ovdelta_f, xs1)
    w2p = jnp.pad(W2, ((0, 128 - NHID), (0, 128 - NCLASS)))
    b1p = jnp.pad(b1, (0, 128 - NHID)).reshape(1, 128)
    b2p = jnp.pad(b2, (0, 128 - NCLASS)).reshape(1, 128)
    gs = _k7(extended_sub_adj, xs1, u1, d, b1p, w2p)
    u2 = _k_urow(128, ovrow_f, ovcol_f, ovdelta_f, gs)
    out = _k9(extended_sub_adj, gs, u2, d, b2p)
    return out[:, :NCLASS]


# speedup vs baseline: 6.9269x; 1.8981x over previous
import jax
import jax.numpy as jnp
from jax import lax
from jax.experimental import pallas as pl
from jax.experimental.pallas import tpu as pltpu
from jax.experimental.pallas import tpu_sc as plsc

N = 4096
NHID = 64
NCLASS = 8
NMASK = 100000
TOPK = 1024
NP = 100352
NROWS = NP // 128
ONE_BITS = 0x3F800000
MAXOV = 1024
HALF_CODES = 1 << 23
HASH_C = -1640531527

NSUB = 16
PAIRS_PER_TILE = NP // NSUB
CHUNK = 128
CHUNKS_PER_TILE = PAIRS_PER_TILE // CHUNK
SC_MESH = dict(core_axis_name="c", subcore_axis_name="s")


def _k1_body(m_ref, src_ref, dst_ref, feat_ref, w1_ref, tk_ref,
             act_ref, code_ref, rcode_ref, x1_ref):
    m = m_ref[...]
    tm = jnp.tanh(m)
    a = jnp.abs(tm)
    ri = lax.broadcasted_iota(jnp.int32, (NROWS, 128), 0)
    ci = lax.broadcasted_iota(jnp.int32, (NROWS, 128), 1)
    gi = ri * 128 + ci
    valid = gi < NMASK
    key = jnp.where(valid, lax.bitcast_convert_type(a, jnp.int32),
                    jnp.int32(-1))
    k = jnp.clip(tk_ref[0, 0], 0, TOPK)

    def vstep(_, lohi):
        lo, hi = lohi
        mid = lax.div(lo + hi, 2)
        c = jnp.sum((key > mid).astype(jnp.int32))
        take_hi = c < k
        return (jnp.where(take_hi, lo, mid + 1), jnp.where(take_hi, mid, hi))

    thr, _ = lax.fori_loop(0, 31, vstep, (jnp.int32(0), jnp.int32(ONE_BITS)))
    eq = key == thr
    n_gt = jnp.sum((key > thr).astype(jnp.int32))
    tn = k - n_gt

    def istep(_, lohi):
        lo2, hi2 = lohi
        mid = lax.div(lo2 + hi2, 2)
        h = jnp.sum((eq & (gi < mid)).astype(jnp.int32))
        ge = h >= tn
        return (jnp.where(ge, lo2, mid + 1), jnp.where(ge, mid, hi2))

    cut, _ = lax.fori_loop(0, 17, istep, (jnp.int32(0), jnp.int32(1 << 17)))
    keep = (key > thr) | (eq & (gi < cut))
    act_ref[...] = jnp.where(
        keep & (tm > 0.5), jnp.int32(1),
        jnp.where(keep & (tm < -0.5), jnp.int32(-1), jnp.int32(0)))
    s = src_ref[...]
    d = dst_ref[...]
    code_ref[...] = lax.shift_left(s, 12) + d
    rcode_ref[...] = lax.shift_left(d, 12) + s
    x1_ref[...] = jnp.dot(feat_ref[...], w1_ref[...],
                          preferred_element_type=jnp.float32)


def _k1(m_p, src_p, dst_p, sub_feat, w1, tk):
    return pl.pallas_call(
        _k1_body,
        in_specs=[pl.BlockSpec(memory_space=pltpu.VMEM)] * 5
        + [pl.BlockSpec(memory_space=pltpu.SMEM)],
        out_specs=[pl.BlockSpec(memory_space=pltpu.VMEM)] * 4,
        out_shape=[
            jax.ShapeDtypeStruct((NROWS, 128), jnp.int32),
            jax.ShapeDtypeStruct((NROWS, 128), jnp.int32),
            jax.ShapeDtypeStruct((NROWS, 128), jnp.int32),
            jax.ShapeDtypeStruct((N, NHID), jnp.float32),
        ],
    )(m_p, src_p, dst_p, sub_feat, w1, tk)


def _k2_body(codes_hbm, rcodes_hbm,
             cntc_hbm, cntr_hbm, dsum_hbm,
             codes_all, rcodes_all, idx_all, val_all, gbuf_all,
             cnts_all, outp, idxd, vald, zbuf, sem,
             cnt_tab, dup_tab):
    c = lax.axis_index("c")
    s = lax.axis_index("s")
    lanes = jnp.arange(16, dtype=jnp.int32)
    tile_base = s * PAIRS_PER_TILE
    nch = CHUNKS_PER_TILE

    def own_word(v, gidx):
        m = (gidx < NMASK) & (lax.shift_right_logical(v, 23) == c)
        w = lax.shift_right_logical(v & (HALF_CODES - 1), 4)
        return m, w


    h_c = pltpu.async_copy(
        codes_hbm.at[pl.ds(tile_base, PAIRS_PER_TILE)], codes_all, sem)
    h_r = pltpu.async_copy(
        rcodes_hbm.at[pl.ds(tile_base, PAIRS_PER_TILE)], rcodes_all, sem)

    def zfill(j, _):
        zbuf[pl.ds(j * 16, 16)] = jnp.zeros((16,), jnp.int32)
        return 0

    lax.fori_loop(0, 1024, zfill, 0)
    for j in range(2):
        pltpu.sync_copy(zbuf, cnt_tab.at[pl.ds(s * 32768 + j * 16384, 16384)])
    pltpu.sync_copy(zbuf.at[pl.ds(0, 4096)], dup_tab.at[pl.ds(s * 4096, 4096)])
    h_c.wait()
    h_r.wait()
    plsc.subcore_barrier()

    @plsc.parallel_loop(0, nch, unroll=4)
    def _s(j):
        for kk in range(8):
            o = j * CHUNK + kk * 16
            v = codes_all[pl.ds(o, 16)]
            gidx = tile_base + o + lanes
            m, w = own_word(v, gidx)
            addv = lax.shift_left(jnp.int32(1), (v & 15) * 2)
            idx_all[pl.ds(o, 16)] = jnp.where(m, w, gidx & ((1 << 19) - 1))
            val_all[pl.ds(o, 16)] = jnp.where(m, addv, 0)

    pltpu.sync_copy(val_all, cnt_tab.at[idx_all], add=True)
    plsc.subcore_barrier()

    pltpu.sync_copy(cnt_tab.at[idx_all], gbuf_all)

    @plsc.parallel_loop(0, nch, unroll=4)
    def _m(j):
        for kk in range(8):
            o = j * CHUNK + kk * 16
            v = codes_all[pl.ds(o, 16)]
            gidx = tile_base + o + lanes
            m, _w = own_word(v, gidx)
            g = gbuf_all[pl.ds(o, 16)]
            cnt = lax.shift_right_logical(g, (v & 15) * 2) & 3
            cnt = jnp.where(m, cnt, 0)
            outp[pl.ds(o, 16)] = cnt
            cnts_all[pl.ds(o, 16)] = cnt
            mdup = m & (cnt >= 2)
            slot = lax.shift_right_logical(v * HASH_C, 16) & 0xFFFF
            idxd[pl.ds(o, 16)] = jnp.where(mdup, slot, gidx & 0xFFFF)
            vald[pl.ds(o, 16)] = jnp.where(mdup, gidx, 0)

    pltpu.sync_copy(outp, cntc_hbm.at[c].at[pl.ds(tile_base, PAIRS_PER_TILE)])
    pltpu.sync_copy(vald, dup_tab.at[idxd], add=True)

    @plsc.parallel_loop(0, nch, unroll=4)
    def _gr(j):
        for kk in range(8):
            o = j * CHUNK + kk * 16
            v = rcodes_all[pl.ds(o, 16)]
            gidx = tile_base + o + lanes
            m, w = own_word(v, gidx)
            idx_all[pl.ds(o, 16)] = jnp.where(m, w, gidx & ((1 << 19) - 1))

    pltpu.sync_copy(cnt_tab.at[idx_all], gbuf_all)

    @plsc.parallel_loop(0, nch, unroll=4)
    def _xr(j):
        for kk in range(8):
            o = j * CHUNK + kk * 16
            v = rcodes_all[pl.ds(o, 16)]
            m, _w = own_word(v, tile_base + o + lanes)
            g = gbuf_all[pl.ds(o, 16)]
            cnt = lax.shift_right_logical(g, (v & 15) * 2) & 3
            outp[pl.ds(o, 16)] = jnp.where(m, cnt, 0)

    pltpu.sync_copy(outp, cntr_hbm.at[c].at[pl.ds(tile_base, PAIRS_PER_TILE)])
    plsc.subcore_barrier()

    pltpu.sync_copy(dup_tab.at[idxd], gbuf_all)

    @plsc.parallel_loop(0, nch, unroll=4)
    def _e(j):
        for kk in range(8):
            o = j * CHUNK + kk * 16
            v = codes_all[pl.ds(o, 16)]
            m, _w = own_word(v, tile_base + o + lanes)
            m = m & (cnts_all[pl.ds(o, 16)] >= 2)
            outp[pl.ds(o, 16)] = jnp.where(m, gbuf_all[pl.ds(o, 16)], 0)

    pltpu.sync_copy(outp, dsum_hbm.at[c].at[pl.ds(tile_base, PAIRS_PER_TILE)])


def _k2(codes, rcodes):
    fn = pl.kernel(
        _k2_body,
        mesh=plsc.VectorSubcoreMesh(**SC_MESH),
        out_type=[
            jax.ShapeDtypeStruct((2, NP), jnp.int32),
            jax.ShapeDtypeStruct((2, NP), jnp.int32),
            jax.ShapeDtypeStruct((2, NP), jnp.int32),
        ],
        scratch_types=[
            pltpu.VMEM((PAIRS_PER_TILE,), jnp.int32),
            pltpu.VMEM((PAIRS_PER_TILE,), jnp.int32),
            pltpu.VMEM((PAIRS_PER_TILE,), jnp.int32),
            pltpu.VMEM((PAIRS_PER_TILE,), jnp.int32),
            pltpu.VMEM((PAIRS_PER_TILE,), jnp.int32),
            pltpu.VMEM((PAIRS_PER_TILE,), jnp.int32),
            pltpu.VMEM((PAIRS_PER_TILE,), jnp.int32),
            pltpu.VMEM((PAIRS_PER_TILE,), jnp.int32),
            pltpu.VMEM((PAIRS_PER_TILE,), jnp.int32),
            pltpu.VMEM((16384,), jnp.int32),
            pltpu.SemaphoreType.DMA,
            pltpu.VMEM_SHARED((1 << 19,), jnp.int32),
            pltpu.VMEM_SHARED((1 << 16,), jnp.int32),
        ],
    )
    return fn(codes, rcodes)


def _k3b_body(adj_ref, out_ref):
    out_ref[...] = jnp.sum(adj_ref[...], axis=1).reshape(1, 1, 128)


def _k3b(adj):
    return pl.pallas_call(
        _k3b_body,
        grid=(32,),
        in_specs=[pl.BlockSpec((128, N), lambda i: (i, 0))],
        out_specs=pl.BlockSpec((1, 1, 128), lambda i: (i, 0, 0)),
        out_shape=jax.ShapeDtypeStruct((32, 1, 128), jnp.float32),
    )(adj)


def _k3a_body(act_ref, code_ref, rcode_ref, cc0_ref, cc1_ref, cr0_ref,
              cr1_ref, ds0_ref, ds1_ref, packed_ref, dest_ref, nn_ref):
    act = act_ref[...]
    cnt = cc0_ref[...] + cc1_ref[...]
    cntr = cr0_ref[...] + cr1_ref[...]
    dsum = ds0_ref[...] + ds1_ref[...]
    ri = lax.broadcasted_iota(jnp.int32, (NROWS, 128), 0)
    ci = lax.broadcasted_iota(jnp.int32, (NROWS, 128), 1)
    gi = ri * 128 + ci
    surv_a = (cnt == 1) | ((cnt == 2) & (2 * gi > dsum))
    cand = act != 0
    tag = lax.shift_left((act > 0).astype(jnp.int32), 24) + jnp.int32(1 << 25)
    surv0 = cand & surv_a & (cntr == 0)
    surv1 = cand & surv_a
    packed_ref[0:NROWS, :] = jnp.where(surv0, code_ref[...] + tag, 0)
    packed_ref[NROWS:2 * NROWS, :] = jnp.where(surv1, rcode_ref[...] + tag, 0)

    tri = (lax.broadcasted_iota(jnp.int32, (NROWS, NROWS), 0)
           > lax.broadcasted_iota(jnp.int32, (NROWS, NROWS), 1)
           ).astype(jnp.float32)
    lt = (lax.broadcasted_iota(jnp.int32, (128, 128), 0)
          <= lax.broadcasted_iota(jnp.int32, (128, 128), 1)
          ).astype(jnp.float32)

    def prefix(sv):
        v = sv.astype(jnp.float32)
        incl = jnp.dot(v, lt, preferred_element_type=jnp.float32)
        rowtot = incl[:, 127:128]
        row_off = jnp.dot(tri, rowtot, preferred_element_type=jnp.float32)
        dest = (row_off + incl - v).astype(jnp.int32)
        n = jnp.sum(sv.astype(jnp.int32))
        return dest, n

    d0, n0 = prefix(surv0)
    d1, n1 = prefix(surv1)
    dest_ref[0:NROWS, :] = d0
    dest_ref[NROWS:2 * NROWS, :] = d1
    nn_ref[0, 0] = n0
    nn_ref[1, 0] = n1


def _k3a(act, code, rcode, cc0, cc1, cr0, cr1, ds0, ds1):
    return pl.pallas_call(
        _k3a_body,
        in_specs=[pl.BlockSpec(memory_space=pltpu.VMEM)] * 9,
        out_specs=[
            pl.BlockSpec(memory_space=pltpu.VMEM),
            pl.BlockSpec(memory_space=pltpu.VMEM),
            pl.BlockSpec(memory_space=pltpu.SMEM),
        ],
        out_shape=[
            jax.ShapeDtypeStruct((2 * NROWS, 128), jnp.int32),
            jax.ShapeDtypeStruct((2 * NROWS, 128), jnp.int32),
            jax.ShapeDtypeStruct((16, 1), jnp.int32),
        ],
    )(act, code, rcode, cc0, cc1, cr0, cr1, ds0, ds1)


def _k4_body(packed_hbm, dest_hbm, nn_hbm, adj_hbm,
             ovrow_hbm, ovcol_hbm, ovdelta_hbm, degd_hbm,
             pall, dall, idx_all, val_all, pvec, idx64, rowb, colb, deltab,
             oldb, nbuf, zbi, zbf, sem, ov_sp, degd_sp):
    c = lax.axis_index("c")
    s = lax.axis_index("s")
    lanes = jnp.arange(16, dtype=jnp.int32)
    tile_base = s * PAIRS_PER_TILE

    for i in range(16):
        zbi[pl.ds(i * 16, 16)] = jnp.zeros((16,), jnp.int32)
        zbf[pl.ds(i * 16, 16)] = jnp.zeros((16,), jnp.float32)
    h_p = pltpu.async_copy(
        packed_hbm.at[c].at[pl.ds(tile_base, PAIRS_PER_TILE)], pall, sem)
    h_d = pltpu.async_copy(
        dest_hbm.at[c].at[pl.ds(tile_base, PAIRS_PER_TILE)], dall, sem)
    pltpu.sync_copy(zbi.at[pl.ds(0, 64)], ov_sp.at[pl.ds(s * 64, 64)])
    pltpu.sync_copy(zbf, degd_sp.at[pl.ds(s * 256, 256)])
    pltpu.sync_copy(nn_hbm, nbuf)
    h_p.wait()
    h_d.wait()
    plsc.subcore_barrier()

    @plsc.parallel_loop(0, CHUNKS_PER_TILE, unroll=4)
    def _p(j):
        for kk in range(8):
            o = j * CHUNK + kk * 16
            v = pall[pl.ds(o, 16)]
            dd = dall[pl.ds(o, 16)]
            m = (lax.shift_right_logical(v, 25) & 1) == 1
            spread = (tile_base + o + lanes) & (MAXOV - 1)
            idx_all[pl.ds(o, 16)] = jnp.where(m, dd, spread)
            val_all[pl.ds(o, 16)] = jnp.where(m, v, 0)
    pltpu.sync_copy(val_all, ov_sp.at[idx_all], add=True)
    plsc.subcore_barrier()

    slot_base = s * 64
    pltpu.sync_copy(ov_sp.at[pl.ds(slot_base, 64)], pvec.at[pl.ds(0, 64)])
    nv = nbuf[pl.ds(0, 16)]
    n_c = jnp.where(c == 0, nv[0], nv[1])
    for kk in range(4):
        v = pvec[pl.ds(kk * 16, 16)]
        live = (slot_base + kk * 16 + lanes) < n_c
        idx64[pl.ds(kk * 16, 16)] = jnp.where(live, v & 0xFFFFFF, 0)
    pltpu.sync_copy(adj_hbm.at[idx64], oldb)
    for kk in range(4):
        v = pvec[pl.ds(kk * 16, 16)]
        live = (slot_base + kk * 16 + lanes) < n_c
        codev = v & 0xFFFFFF
        newv = (lax.shift_right_logical(v, 24) & 1).astype(jnp.float32)
        delta = jnp.where(live, newv - oldb[pl.ds(kk * 16, 16)], 0.0)
        rowb[pl.ds(kk * 16, 16)] = jnp.where(
            live, lax.shift_right_logical(codev, 12), 0)
        colb[pl.ds(kk * 16, 16)] = jnp.where(live, codev & 4095, 0)
        deltab[pl.ds(kk * 16, 16)] = delta
    pltpu.sync_copy(rowb, ovrow_hbm.at[c].at[pl.ds(slot_base, 64)])
    pltpu.sync_copy(colb, ovcol_hbm.at[c].at[pl.ds(slot_base, 64)])
    pltpu.sync_copy(deltab, ovdelta_hbm.at[c].at[pl.ds(slot_base, 64)])
    pltpu.sync_copy(deltab, degd_sp.at[rowb], add=True)
    plsc.subcore_barrier()

    pltpu.sync_copy(degd_sp.at[pl.ds(s * 256, 256)],
                    degd_hbm.at[c].at[pl.ds(s * 256, 256)])


def _k4(packed, dest, nn, adj_flat):
    fn = pl.kernel(
        _k4_body,
        mesh=plsc.VectorSubcoreMesh(**SC_MESH),
        out_type=[
            jax.ShapeDtypeStruct((2, MAXOV), jnp.int32),
            jax.ShapeDtypeStruct((2, MAXOV), jnp.int32),
            jax.ShapeDtypeStruct((2, MAXOV), jnp.float32),
            jax.ShapeDtypeStruct((2, N), jnp.float32),
        ],
        scratch_types=[
            pltpu.VMEM((PAIRS_PER_TILE,), jnp.int32),
            pltpu.VMEM((PAIRS_PER_TILE,), jnp.int32),
            pltpu.VMEM((PAIRS_PER_TILE,), jnp.int32),
            pltpu.VMEM((PAIRS_PER_TILE,), jnp.int32),
            pltpu.VMEM((CHUNK,), jnp.int32),
            pltpu.VMEM((64,), jnp.int32),
            pltpu.VMEM((64,), jnp.int32),
            pltpu.VMEM((64,), jnp.int32),
            pltpu.VMEM((64,), jnp.float32),
            pltpu.VMEM((64,), jnp.float32),
            pltpu.VMEM((16,), jnp.int32),
            pltpu.VMEM((256,), jnp.int32),
            pltpu.VMEM((256,), jnp.float32),
            pltpu.SemaphoreType.DMA,
            pltpu.VMEM_SHARED((MAXOV,), jnp.int32),
            pltpu.VMEM_SHARED((N,), jnp.float32),
        ],
    )
    return fn(packed, dest, nn, adj_flat)


def _k5_body(bd_ref, dd0_ref, dd1_ref, x1_ref, d_ref, xs1_ref):
    deg = bd_ref[...] + dd0_ref[...] + dd1_ref[...] + 1.0
    d = lax.rsqrt(jnp.maximum(deg, 1e-12))
    d_ref[...] = d
    xs1_ref[:, 0:NHID] = d * x1_ref[...]
    xs1_ref[:, NHID:128] = jnp.zeros((N, 128 - NHID), jnp.float32)


def _k5(base_deg, dd0, dd1, x1):
    return pl.pallas_call(
        _k5_body,
        in_specs=[pl.BlockSpec(memory_space=pltpu.VMEM)] * 4,
        out_specs=[pl.BlockSpec(memory_space=pltpu.VMEM)] * 2,
        out_shape=[
            jax.ShapeDtypeStruct((N, 1), jnp.float32),
            jax.ShapeDtypeStruct((N, 128), jnp.float32),
        ],
    )(base_deg, dd0, dd1, x1)


def _urow_body(width, ovrow_hbm, ovcol_hbm, ovdelta_hbm, tab_hbm, out_hbm,
               rowb, colb, deltab, idxb, rows_v, zrow, sem, u_sp):
    c = lax.axis_index("c")
    s = lax.axis_index("s")
    lanes = jnp.arange(16, dtype=jnp.int32)
    ebase = s * 128
    row_lo = c * (N // 2)

    h_r = pltpu.async_copy(ovrow_hbm.at[pl.ds(ebase, 128)], rowb, sem)
    h_g = pltpu.async_copy(ovcol_hbm.at[pl.ds(ebase, 128)], colb, sem)
    h_d = pltpu.async_copy(ovdelta_hbm.at[pl.ds(ebase, 128)], deltab, sem)
    for i in range(16):
        for kk in range(width // 16):
            zrow[i, pl.ds(kk * 16, 16)] = jnp.zeros((16,), jnp.float32)

    @plsc.parallel_loop(0, (N // 2 // NSUB) // 16, unroll=2)
    def _z(j):
        pltpu.sync_copy(zrow, u_sp.at[pl.ds(s * (N // 2 // NSUB) + j * 16, 16)])

    h_r.wait()
    h_g.wait()
    h_d.wait()
    plsc.subcore_barrier()

    pltpu.sync_copy(tab_hbm.at[colb], rows_v)
    dm_all = []
    for q in range(8):
        rv = rowb[pl.ds(q * 16, 16)]
        m = lax.shift_right_logical(rv, 11) == c
        spread = (ebase + q * 16 + lanes) & (N // 2 - 1)
        idxb[pl.ds(q * 16, 16)] = jnp.where(m, rv - row_lo, spread)
        dm_all.append(jnp.where(m, deltab[pl.ds(q * 16, 16)], 0.0))
    for i in range(128):
        dsc = dm_all[i // 16][i % 16]
        for kk in range(width // 16):
            rows_v[i, pl.ds(kk * 16, 16)] = rows_v[i, pl.ds(kk * 16, 16)] * dsc
    pltpu.sync_copy(rows_v, u_sp.at[idxb], add=True)
    plsc.subcore_barrier()

    rpt = N // 2 // NSUB
    pltpu.sync_copy(u_sp.at[pl.ds(s * rpt, rpt)],
                    out_hbm.at[c].at[pl.ds(s * rpt, rpt)])


def _k_urow(width, ovrow, ovcol, ovdelta, tab):
    def body(*refs):
        _urow_body(width, *refs)

    fn = pl.kernel(
        body,
        mesh=plsc.VectorSubcoreMesh(**SC_MESH),
        out_type=[jax.ShapeDtypeStruct((2, N // 2, width), jnp.float32)],
        scratch_types=[
            pltpu.VMEM((128,), jnp.int32),
            pltpu.VMEM((128,), jnp.int32),
            pltpu.VMEM((128,), jnp.float32),
            pltpu.VMEM((128,), jnp.int32),
            pltpu.VMEM((128, width), jnp.float32),
            pltpu.VMEM((16, width), jnp.float32),
            pltpu.SemaphoreType.DMA,
            pltpu.VMEM_SHARED((N // 2, width), jnp.float32),
        ],
    )
    return fn(ovrow, ovcol, ovdelta, tab)[0]


def _k7_body(adj_ref, xs1_ref, u1_ref, d_ref, b1_ref, w2_ref, gs_ref):
    i = pl.program_id(0)
    y = jnp.dot(adj_ref[...], xs1_ref[...], preferred_element_type=jnp.float32)
    rows = xs1_ref[pl.ds(i * 256, 256), :]
    t = d_ref[...] * (y + u1_ref[0] + rows) + b1_ref[...]
    h = jnp.maximum(t, 0.0)
    gs_ref[...] = d_ref[...] * jnp.dot(h, w2_ref[...],
                                       preferred_element_type=jnp.float32)


def _k7(adj, xs1, u1, d, b1, w2p):
    return pl.pallas_call(
        _k7_body,
        grid=(16,),
        in_specs=[
            pl.BlockSpec((256, N), lambda i: (i, 0)),
            pl.BlockSpec((N, 128), lambda i: (0, 0)),
            pl.BlockSpec((1, 256, 128), lambda i: (i // 8, i % 8, 0)),
            pl.BlockSpec((256, 1), lambda i: (i, 0)),
            pl.BlockSpec((1, 128), lambda i: (0, 0)),
            pl.BlockSpec((128, 128), lambda i: (0, 0)),
        ],
        out_specs=pl.BlockSpec((256, 128), lambda i: (i, 0)),
        out_shape=jax.ShapeDtypeStruct((N, 128), jnp.float32),
    )(adj, xs1, u1, d, b1, w2p)


def _k9_body(adj_ref, gs_ref, u2_ref, d_ref, b2_ref, out_ref):
    i = pl.program_id(0)
    y = jnp.dot(adj_ref[...], gs_ref[...], preferred_element_type=jnp.float32)
    rows = gs_ref[pl.ds(i * 256, 256), :]
    out_ref[...] = d_ref[...] * (y + u2_ref[0] + rows) + b2_ref[...]


def _k9(adj, gs, u2, d, b2p):
    return pl.pallas_call(
        _k9_body,
        grid=(16,),
        in_specs=[
            pl.BlockSpec((256, N), lambda i: (i, 0)),
            pl.BlockSpec((N, 128), lambda i: (0, 0)),
            pl.BlockSpec((1, 256, 128), lambda i: (i // 8, i % 8, 0)),
            pl.BlockSpec((256, 1), lambda i: (i, 0)),
            pl.BlockSpec((1, 128), lambda i: (0, 0)),
        ],
        out_specs=pl.BlockSpec((256, 128), lambda i: (i, 0)),
        out_shape=jax.ShapeDtypeStruct((N, 128), jnp.float32),
    )(adj, gs, u2, d, b2p)


def kernel(extended_sub_adj, sub_feat, M, pairs, W1, b1, W2, b2, top_k):
    pad = NP - NMASK
    m_p = jnp.pad(M, (0, pad)).reshape(NROWS, 128)
    src = jnp.pad(pairs[:, 0], (0, pad)).reshape(NROWS, 128)
    dst = jnp.pad(pairs[:, 1], (0, pad)).reshape(NROWS, 128)
    tk = jnp.asarray(top_k, jnp.int32).reshape(1, 1)

    act, code, rcode, x1 = _k1(m_p, src, dst, sub_feat, W1, tk)
    cntc, cntr, dsum = _k2(code.reshape(NP), rcode.reshape(NP))
    base_deg = _k3b(extended_sub_adj).reshape(N, 1)
    packed, dest, nn = _k3a(
        act, code, rcode,
        cntc[0].reshape(NROWS, 128), cntc[1].reshape(NROWS, 128),
        cntr[0].reshape(NROWS, 128), cntr[1].reshape(NROWS, 128),
        dsum[0].reshape(NROWS, 128), dsum[1].reshape(NROWS, 128))
    ovrow, ovcol, ovdelta, degd = _k4(
        packed.reshape(2, NP), dest.reshape(2, NP), nn.reshape(16),
        extended_sub_adj.reshape(N * N))
    d, xs1 = _k5(base_deg, degd[0].reshape(N, 1), degd[1].reshape(N, 1), x1)
    ovrow_f = ovrow.reshape(2 * MAXOV)
    ovcol_f = ovcol.reshape(2 * MAXOV)
    ovdelta_f = ovdelta.reshape(2 * MAXOV)
    u1 = _k_urow(128, ovrow_f, ovcol_f, ovdelta_f, xs1)
    w2p = jnp.pad(W2, ((0, 128 - NHID), (0, 128 - NCLASS)))
    b1p = jnp.pad(b1, (0, 128 - NHID)).reshape(1, 128)
    b2p = jnp.pad(b2, (0, 128 - NCLASS)).reshape(1, 128)
    gs = _k7(extended_sub_adj, xs1, u1, d, b1p, w2p)
    u2 = _k_urow(128, ovrow_f, ovcol_f, ovdelta_f, gs)
    out = _k9(extended_sub_adj, gs, u2, d, b2p)
    return out[:, :NCLASS]
